# baseline (speedup 1.0000x reference)
import functools
import jax
import jax.numpy as jnp
from jax import lax
from jax.experimental import pallas as pl
from jax.experimental.pallas import tpu as pltpu
from jax.experimental.pallas import tpu_sc as plsc

_N = 10000
_E = 320000
_IN = 128
_H = 256
_OUT = 128

_NC = 2
_NS = 16
_NW = _NC * _NS
_C = 128
_NCH = 80
_EPT = _NCH * _C
_EPAD = _NW * _EPT
_SBCH = 4
_NSB = _NCH // _SBCH
_NA = 10112
_RPT = _NA // _NS
_ROW_CHUNKS = tuple((r0, min(_C, _RPT - r0)) for r0 in range(0, _RPT, _C))
_F = 128


def _sc_feat_body(table, src4, dst4, ewf, out_s,
                  src_v, dst_v, ew_v, rows0, rows1, acc, semA, semB):
  c = lax.axis_index("c")
  s = lax.axis_index("s")
  wid = s * _NC + c
  t0 = s * _RPT
  ebase = wid * _EPT

  zeros16 = jnp.zeros((16,), jnp.float32)

  def zbody(r, carry):
    for d in range(_F // 16):
      rows0[r, pl.ds(d * 16, 16)] = zeros16
    return carry
  lax.fori_loop(0, _C, zbody, 0)

  for r0, rn in _ROW_CHUNKS:
    pltpu.sync_copy(rows0.at[pl.ds(0, rn)], acc.at[pl.ds(t0 + r0, rn)])
  plsc.subcore_barrier()

  rows = (rows0, rows1)
  sems = (semA, semB)

  def scale(rr, k):
    def body(e, cc):
      wv = ew_v[pl.ds((k * _C + e) * 16, 16)]
      for d in range(_F // 16):
        sl = pl.ds(d * 16, 16)
        rr[e, sl] = rr[e, sl] * wv
      return cc
    lax.fori_loop(0, _C, body, 0)

  def sb_body(b, carry):
    sb0 = b * _SBCH
    pltpu.sync_copy(src4.at[wid, b], src_v)
    pltpu.sync_copy(dst4.at[wid, b], dst_v)
    pltpu.sync_copy(ewf.at[pl.ds((ebase + sb0 * _C) * 16, _SBCH * _C * 16)],
                    ew_v)
    pltpu.async_copy(table.at[src_v.at[0]], rows0, semA)
    for k in range(_SBCH):
      if k + 1 < _SBCH:
        pltpu.async_copy(table.at[src_v.at[k + 1]],
                         rows[(k + 1) % 2], sems[(k + 1) % 2])
      pltpu.make_async_copy(table.at[src_v.at[k]],
                            rows[k % 2], sems[k % 2]).wait()
      scale(rows[k % 2], k)
      pltpu.sync_copy(rows[k % 2], acc.at[dst_v.at[k]], add=True)
    return carry
  lax.fori_loop(0, _NSB, sb_body, 0)
  plsc.subcore_barrier()

  for r0, rn in _ROW_CHUNKS:
    rb = t0 + r0
    pltpu.sync_copy(acc.at[pl.ds(rb, rn)], out_s.at[c, pl.ds(rb, rn)])


def _make_sc_feat():
  mesh = plsc.VectorSubcoreMesh(core_axis_name="c", subcore_axis_name="s")
  out_type = jax.ShapeDtypeStruct((_NC, _NA, _F), jnp.float32)
  scratch = [
      pltpu.VMEM((_SBCH, _C), jnp.int32),
      pltpu.VMEM((_SBCH, _C), jnp.int32),
      pltpu.VMEM((_SBCH * _C * 16,), jnp.float32),
      pltpu.VMEM((_C, _F), jnp.float32),
      pltpu.VMEM((_C, _F), jnp.float32),
      pltpu.VMEM_SHARED((_NA, _F), jnp.float32),
      pltpu.SemaphoreType.DMA,
      pltpu.SemaphoreType.DMA,
  ]
  return pl.kernel(_sc_feat_body, out_type=out_type, mesh=mesh,
                   scratch_types=scratch)


def _sc_deg_body(dst3, out_d, dst_v, ones_v, z16_v, dacc):
  c = lax.axis_index("c")
  s = lax.axis_index("s")
  wid = s * _NC + c
  t0 = s * _RPT

  zeros16 = jnp.zeros((16,), jnp.float32)
  ones16 = jnp.ones((16,), jnp.float32)

  def zbody(r, carry):
    ones_v[r, :] = ones16
    z16_v[r, :] = zeros16
    return carry
  lax.fori_loop(0, _C, zbody, 0)

  for r0, rn in _ROW_CHUNKS:
    pltpu.sync_copy(z16_v.at[pl.ds(0, rn)], dacc.at[pl.ds(t0 + r0, rn)])
  plsc.subcore_barrier()

  pltpu.sync_copy(dst3.at[wid], dst_v)

  def chunk(j, carry):
    pltpu.sync_copy(ones_v, dacc.at[dst_v.at[j]], add=True)
    return carry
  lax.fori_loop(0, _NCH, chunk, 0)
  plsc.subcore_barrier()

  for r0, rn in _ROW_CHUNKS:
    rb = t0 + r0
    pltpu.sync_copy(dacc.at[pl.ds(rb, rn)], out_d.at[c, pl.ds(rb, rn)])


def _make_sc_deg():
  mesh = plsc.VectorSubcoreMesh(core_axis_name="c", subcore_axis_name="s")
  out_type = jax.ShapeDtypeStruct((_NC, _NA, 16), jnp.float32)
  scratch = [
      pltpu.VMEM((_NCH, _C), jnp.int32),
      pltpu.VMEM((_C, 16), jnp.float32),
      pltpu.VMEM((_C, 16), jnp.float32),
      pltpu.VMEM_SHARED((_NA, 16), jnp.float32),
  ]
  return pl.kernel(_sc_deg_body, out_type=out_type, mesh=mesh,
                   scratch_types=scratch,
                   compiler_params=pltpu.CompilerParams(
                       use_tc_tiling_on_sc=False))


_sc_feat = _make_sc_feat()
_sc_degree = _make_sc_deg()


def _norm_body(w_ref, o_ref):
  w = w_ref[...]
  mn = jnp.min(w)
  mx = jnp.max(w)
  o_ref[...] = jnp.where(mx == mn, jnp.ones_like(w), (w - mn) / (mx - mn))


_norm = pl.pallas_call(
    _norm_body,
    out_shape=jax.ShapeDtypeStruct((_E // 128, 128), jnp.float32))


_BLK = 400
_NBLK = _N // _BLK


def _dense0_body(sp, dp, x, wl0, bl0, wr0, gamma, beta, rm, rv, wl1,
                 h_out, g_out):
  ssum = sp[0] + sp[1]
  dsum = dp[0] + dp[1]
  deg = jnp.clip(dsum[:, 0:1], 1.0, None)
  aggr = ssum / deg
  dn = (((1,), (1,)), ((), ()))
  pre = (lax.dot_general(aggr, wl0[...], dn, preferred_element_type=jnp.float32)
         + bl0[...]
         + lax.dot_general(x[...], wr0[...], dn, preferred_element_type=jnp.float32))
  inv = lax.rsqrt(rv[...] + 1e-5)
  hh = jnp.maximum((pre - rm[...]) * inv * gamma[...] + beta[...], 0.0)
  h_out[...] = hh
  g_out[...] = lax.dot_general(hh, wl1[...], dn, preferred_element_type=jnp.float32)


_dense0 = pl.pallas_call(
    _dense0_body,
    grid=(_NBLK,),
    in_specs=[
        pl.BlockSpec((_NC, _BLK, _F), lambda i: (0, i, 0)),
        pl.BlockSpec((_NC, _BLK, 16), lambda i: (0, i, 0)),
        pl.BlockSpec((_BLK, _IN), lambda i: (i, 0)),
        pl.BlockSpec((_H, _IN), lambda i: (0, 0)),
        pl.BlockSpec((1, _H), lambda i: (0, 0)),
        pl.BlockSpec((_H, _IN), lambda i: (0, 0)),
        pl.BlockSpec((1, _H), lambda i: (0, 0)),
        pl.BlockSpec((1, _H), lambda i: (0, 0)),
        pl.BlockSpec((1, _H), lambda i: (0, 0)),
        pl.BlockSpec((1, _H), lambda i: (0, 0)),
        pl.BlockSpec((_OUT, _H), lambda i: (0, 0)),
    ],
    out_specs=[
        pl.BlockSpec((_BLK, _H), lambda i: (i, 0)),
        pl.BlockSpec((_BLK, _OUT), lambda i: (i, 0)),
    ],
    out_shape=[
        jax.ShapeDtypeStruct((_N, _H), jnp.float32),
        jax.ShapeDtypeStruct((_N, _OUT), jnp.float32),
    ])


def _dense1_body(sp, dp, h, wr1, bl1, lsm_out, var_out, acc_s):
  i = pl.program_id(0)
  ssum = sp[0] + sp[1]
  dsum = dp[0] + dp[1]
  deg = jnp.clip(dsum[:, 0:1], 1.0, None)
  dn = (((1,), (1,)), ((), ()))
  o = (ssum / deg + bl1[...]
       + lax.dot_general(h[...], wr1[...], dn, preferred_element_type=jnp.float32))
  m = jnp.max(o, axis=1, keepdims=True)
  lse = jnp.log(jnp.sum(jnp.exp(o - m), axis=1, keepdims=True)) + m
  lsm_out[...] = o - lse
  bs = jnp.sum(o)
  bss = jnp.sum(o * o)

  @pl.when(i == 0)
  def _():
    acc_s[0] = bs
    acc_s[1] = bss

  @pl.when(i > 0)
  def _():
    acc_s[0] = acc_s[0] + bs
    acc_s[1] = acc_s[1] + bss

  tot = float(_N * _OUT)
  var_out[...] = jnp.full((1, 1), (acc_s[1] - acc_s[0] * acc_s[0] / tot)
                          / (tot - 1.0), jnp.float32)


_dense1 = pl.pallas_call(
    _dense1_body,
    grid=(_NBLK,),
    in_specs=[
        pl.BlockSpec((_NC, _BLK, _OUT), lambda i: (0, i, 0)),
        pl.BlockSpec((_NC, _BLK, 16), lambda i: (0, i, 0)),
        pl.BlockSpec((_BLK, _H), lambda i: (i, 0)),
        pl.BlockSpec((_OUT, _H), lambda i: (0, 0)),
        pl.BlockSpec((1, _OUT), lambda i: (0, 0)),
    ],
    out_specs=[
        pl.BlockSpec((_BLK, _OUT), lambda i: (i, 0)),
        pl.BlockSpec((1, 1), lambda i: (0, 0)),
    ],
    out_shape=[
        jax.ShapeDtypeStruct((_N, _OUT), jnp.float32),
        jax.ShapeDtypeStruct((1, 1), jnp.float32),
    ],
    scratch_shapes=[pltpu.SMEM((2,), jnp.float32)])


def kernel(x, edge_index, edge_weight, Wl0, bl0, Wr0, gamma, beta,
           running_mean, running_var, Wl1, bl1, Wr1):
  ewn = _norm(edge_weight.reshape(_E // 128, 128)).reshape(_E)
  pad = _EPAD - _E
  src4 = jnp.concatenate([edge_index[0], jnp.zeros((pad,), jnp.int32)]
                         ).reshape(_NW, _NSB, _SBCH, _C)
  junk = _N + jnp.arange(pad, dtype=jnp.int32) % (_NA - _N)
  dst_p = jnp.concatenate([edge_index[1], junk])
  dst3 = dst_p.reshape(_NW, _NCH, _C)
  dst4 = dst_p.reshape(_NW, _NSB, _SBCH, _C)
  ewn_p = jnp.concatenate([ewn, jnp.zeros((pad,), jnp.float32)])
  ewf = jnp.broadcast_to(ewn_p[:, None], (_EPAD, 16)).reshape(_EPAD * 16)

  dp0 = _sc_degree(dst3)
  sp0 = _sc_feat(x, src4, dst4, ewf)
  h, g = _dense0(sp0, dp0, x, Wl0, bl0.reshape(1, -1), Wr0,
                 gamma.reshape(1, -1), beta.reshape(1, -1),
                 running_mean.reshape(1, -1), running_var.reshape(1, -1), Wl1)
  sp1 = _sc_feat(g, src4, dst4, ewf)
  lsm, var = _dense1(sp1, dp0, h, Wr1, bl1.reshape(1, -1))
  return lsm, var.reshape(())


# speedup vs baseline: 1.0019x; 1.0019x over previous
import functools
import jax
import jax.numpy as jnp
from jax import lax
from jax.experimental import pallas as pl
from jax.experimental.pallas import tpu as pltpu
from jax.experimental.pallas import tpu_sc as plsc

_N = 10000
_E = 320000
_IN = 128
_H = 256
_OUT = 128

_NC = 2
_NS = 16
_NW = _NC * _NS
_C = 128
_NCH = 80
_EPT = _NCH * _C
_EPAD = _NW * _EPT
_SBCH = 4
_NSB = _NCH // _SBCH
_TSB = _NW * _NSB
_NSB0 = 20
_NSB1 = _NSB * 2 - _NSB0
_NA = 10112
_RPT = _NA // _NS
_ROW_CHUNKS = tuple((r0, min(_C, _RPT - r0)) for r0 in range(0, _RPT, _C))
_F = 128


def _sc_feat_body(table, src4, dst4, ewf, out_s,
                  src_v, dst_v, ew_v, rows0, rows1, acc, semA, semB):
  c = lax.axis_index("c")
  s = lax.axis_index("s")
  t0 = s * _RPT
  nsb = jnp.where(c == 0, _NSB0, _NSB1)
  sbbase = jnp.where(c == 0, s * _NSB0, _NS * _NSB0 + s * _NSB1)

  zeros16 = jnp.zeros((16,), jnp.float32)

  def zbody(r, carry):
    for d in range(_F // 16):
      rows0[r, pl.ds(d * 16, 16)] = zeros16
    return carry
  lax.fori_loop(0, _C, zbody, 0)

  for r0, rn in _ROW_CHUNKS:
    pltpu.sync_copy(rows0.at[pl.ds(0, rn)], acc.at[pl.ds(t0 + r0, rn)])
  plsc.subcore_barrier()

  rows = (rows0, rows1)
  sems = (semA, semB)

  def scale(rr, k):
    def body(e, cc):
      wv = ew_v[pl.ds((k * _C + e) * 16, 16)]
      for d in range(_F // 16):
        sl = pl.ds(d * 16, 16)
        rr[e, sl] = rr[e, sl] * wv
      return cc
    lax.fori_loop(0, _C, body, 0)

  def sb_body(b, carry):
    sbg = sbbase + b
    pltpu.sync_copy(src4.at[sbg], src_v)
    pltpu.sync_copy(dst4.at[sbg], dst_v)
    pltpu.sync_copy(ewf.at[pl.ds(sbg * (_SBCH * _C * 16), _SBCH * _C * 16)],
                    ew_v)
    pltpu.async_copy(table.at[src_v.at[0]], rows0, semA)
    for k in range(_SBCH):
      if k + 1 < _SBCH:
        pltpu.async_copy(table.at[src_v.at[k + 1]],
                         rows[(k + 1) % 2], sems[(k + 1) % 2])
      pltpu.make_async_copy(table.at[src_v.at[k]],
                            rows[k % 2], sems[k % 2]).wait()
      scale(rows[k % 2], k)
      pltpu.sync_copy(rows[k % 2], acc.at[dst_v.at[k]], add=True)
    return carry
  lax.fori_loop(0, nsb, sb_body, 0)
  plsc.subcore_barrier()

  for r0, rn in _ROW_CHUNKS:
    rb = t0 + r0
    pltpu.sync_copy(acc.at[pl.ds(rb, rn)], out_s.at[c, pl.ds(rb, rn)])


def _make_sc_feat():
  mesh = plsc.VectorSubcoreMesh(core_axis_name="c", subcore_axis_name="s")
  out_type = jax.ShapeDtypeStruct((_NC, _NA, _F), jnp.float32)
  scratch = [
      pltpu.VMEM((_SBCH, _C), jnp.int32),
      pltpu.VMEM((_SBCH, _C), jnp.int32),
      pltpu.VMEM((_SBCH * _C * 16,), jnp.float32),
      pltpu.VMEM((_C, _F), jnp.float32),
      pltpu.VMEM((_C, _F), jnp.float32),
      pltpu.VMEM_SHARED((_NA, _F), jnp.float32),
      pltpu.SemaphoreType.DMA,
      pltpu.SemaphoreType.DMA,
  ]
  return pl.kernel(_sc_feat_body, out_type=out_type, mesh=mesh,
                   scratch_types=scratch)


def _sc_deg_body(dst3, out_d, dst_v, ones_v, z16_v, dacc):
  c = lax.axis_index("c")
  s = lax.axis_index("s")
  wid = s * _NC + c
  t0 = s * _RPT

  zeros16 = jnp.zeros((16,), jnp.float32)
  ones16 = jnp.ones((16,), jnp.float32)

  def zbody(r, carry):
    ones_v[r, :] = ones16
    z16_v[r, :] = zeros16
    return carry
  lax.fori_loop(0, _C, zbody, 0)

  for r0, rn in _ROW_CHUNKS:
    pltpu.sync_copy(z16_v.at[pl.ds(0, rn)], dacc.at[pl.ds(t0 + r0, rn)])
  plsc.subcore_barrier()

  pltpu.sync_copy(dst3.at[wid], dst_v)

  def chunk(j, carry):
    pltpu.sync_copy(ones_v, dacc.at[dst_v.at[j]], add=True)
    return carry
  lax.fori_loop(0, _NCH, chunk, 0)
  plsc.subcore_barrier()

  for r0, rn in _ROW_CHUNKS:
    rb = t0 + r0
    pltpu.sync_copy(dacc.at[pl.ds(rb, rn)], out_d.at[c, pl.ds(rb, rn)])


def _make_sc_deg():
  mesh = plsc.VectorSubcoreMesh(core_axis_name="c", subcore_axis_name="s")
  out_type = jax.ShapeDtypeStruct((_NC, _NA, 16), jnp.float32)
  scratch = [
      pltpu.VMEM((_NCH, _C), jnp.int32),
      pltpu.VMEM((_C, 16), jnp.float32),
      pltpu.VMEM((_C, 16), jnp.float32),
      pltpu.VMEM_SHARED((_NA, 16), jnp.float32),
  ]
  return pl.kernel(_sc_deg_body, out_type=out_type, mesh=mesh,
                   scratch_types=scratch,
                   compiler_params=pltpu.CompilerParams(
                       use_tc_tiling_on_sc=False))


_sc_feat = _make_sc_feat()
_sc_degree = _make_sc_deg()


def _norm_body(w_ref, o_ref):
  w = w_ref[...]
  mn = jnp.min(w)
  mx = jnp.max(w)
  o_ref[...] = jnp.where(mx == mn, jnp.ones_like(w), (w - mn) / (mx - mn))


_norm = pl.pallas_call(
    _norm_body,
    out_shape=jax.ShapeDtypeStruct((_E // 128, 128), jnp.float32))


_BLK = 400
_NBLK = _N // _BLK


def _dense0_body(sp, dp, x, wl0, bl0, wr0, gamma, beta, rm, rv, wl1,
                 h_out, g_out):
  ssum = sp[0] + sp[1]
  dsum = dp[0] + dp[1]
  deg = jnp.clip(dsum[:, 0:1], 1.0, None)
  aggr = ssum / deg
  dn = (((1,), (1,)), ((), ()))
  pre = (lax.dot_general(aggr, wl0[...], dn, preferred_element_type=jnp.float32)
         + bl0[...]
         + lax.dot_general(x[...], wr0[...], dn, preferred_element_type=jnp.float32))
  inv = lax.rsqrt(rv[...] + 1e-5)
  hh = jnp.maximum((pre - rm[...]) * inv * gamma[...] + beta[...], 0.0)
  h_out[...] = hh
  g_out[...] = lax.dot_general(hh, wl1[...], dn, preferred_element_type=jnp.float32)


_dense0 = pl.pallas_call(
    _dense0_body,
    grid=(_NBLK,),
    in_specs=[
        pl.BlockSpec((_NC, _BLK, _F), lambda i: (0, i, 0)),
        pl.BlockSpec((_NC, _BLK, 16), lambda i: (0, i, 0)),
        pl.BlockSpec((_BLK, _IN), lambda i: (i, 0)),
        pl.BlockSpec((_H, _IN), lambda i: (0, 0)),
        pl.BlockSpec((1, _H), lambda i: (0, 0)),
        pl.BlockSpec((_H, _IN), lambda i: (0, 0)),
        pl.BlockSpec((1, _H), lambda i: (0, 0)),
        pl.BlockSpec((1, _H), lambda i: (0, 0)),
        pl.BlockSpec((1, _H), lambda i: (0, 0)),
        pl.BlockSpec((1, _H), lambda i: (0, 0)),
        pl.BlockSpec((_OUT, _H), lambda i: (0, 0)),
    ],
    out_specs=[
        pl.BlockSpec((_BLK, _H), lambda i: (i, 0)),
        pl.BlockSpec((_BLK, _OUT), lambda i: (i, 0)),
    ],
    out_shape=[
        jax.ShapeDtypeStruct((_N, _H), jnp.float32),
        jax.ShapeDtypeStruct((_N, _OUT), jnp.float32),
    ])


def _dense1_body(sp, dp, h, wr1, bl1, lsm_out, var_out, acc_s):
  i = pl.program_id(0)
  ssum = sp[0] + sp[1]
  dsum = dp[0] + dp[1]
  deg = jnp.clip(dsum[:, 0:1], 1.0, None)
  dn = (((1,), (1,)), ((), ()))
  o = (ssum / deg + bl1[...]
       + lax.dot_general(h[...], wr1[...], dn, preferred_element_type=jnp.float32))
  m = jnp.max(o, axis=1, keepdims=True)
  lse = jnp.log(jnp.sum(jnp.exp(o - m), axis=1, keepdims=True)) + m
  lsm_out[...] = o - lse
  bs = jnp.sum(o)
  bss = jnp.sum(o * o)

  @pl.when(i == 0)
  def _():
    acc_s[0] = bs
    acc_s[1] = bss

  @pl.when(i > 0)
  def _():
    acc_s[0] = acc_s[0] + bs
    acc_s[1] = acc_s[1] + bss

  tot = float(_N * _OUT)
  var_out[...] = jnp.full((1, 1), (acc_s[1] - acc_s[0] * acc_s[0] / tot)
                          / (tot - 1.0), jnp.float32)


_dense1 = pl.pallas_call(
    _dense1_body,
    grid=(_NBLK,),
    in_specs=[
        pl.BlockSpec((_NC, _BLK, _OUT), lambda i: (0, i, 0)),
        pl.BlockSpec((_NC, _BLK, 16), lambda i: (0, i, 0)),
        pl.BlockSpec((_BLK, _H), lambda i: (i, 0)),
        pl.BlockSpec((_OUT, _H), lambda i: (0, 0)),
        pl.BlockSpec((1, _OUT), lambda i: (0, 0)),
    ],
    out_specs=[
        pl.BlockSpec((_BLK, _OUT), lambda i: (i, 0)),
        pl.BlockSpec((1, 1), lambda i: (0, 0)),
    ],
    out_shape=[
        jax.ShapeDtypeStruct((_N, _OUT), jnp.float32),
        jax.ShapeDtypeStruct((1, 1), jnp.float32),
    ],
    scratch_shapes=[pltpu.SMEM((2,), jnp.float32)])


def kernel(x, edge_index, edge_weight, Wl0, bl0, Wr0, gamma, beta,
           running_mean, running_var, Wl1, bl1, Wr1):
  ewn = _norm(edge_weight.reshape(_E // 128, 128)).reshape(_E)
  pad = _EPAD - _E
  src4 = jnp.concatenate([edge_index[0], jnp.zeros((pad,), jnp.int32)]
                         ).reshape(_TSB, _SBCH, _C)
  junk = _N + jnp.arange(pad, dtype=jnp.int32) % (_NA - _N)
  dst_p = jnp.concatenate([edge_index[1], junk])
  dst3 = dst_p.reshape(_NW, _NCH, _C)
  dst4 = dst_p.reshape(_TSB, _SBCH, _C)
  ewn_p = jnp.concatenate([ewn, jnp.zeros((pad,), jnp.float32)])
  ewf = jnp.broadcast_to(ewn_p[:, None], (_EPAD, 16)).reshape(_EPAD * 16)

  dp0 = _sc_degree(dst3)
  sp0 = _sc_feat(x, src4, dst4, ewf)
  h, g = _dense0(sp0, dp0, x, Wl0, bl0.reshape(1, -1), Wr0,
                 gamma.reshape(1, -1), beta.reshape(1, -1),
                 running_mean.reshape(1, -1), running_var.reshape(1, -1), Wl1)
  sp1 = _sc_feat(g, src4, dst4, ewf)
  lsm, var = _dense1(sp1, dp0, h, Wr1, bl1.reshape(1, -1))
  return lsm, var.reshape(())


# speedup vs baseline: 1.1435x; 1.1414x over previous
import functools
import jax
import jax.numpy as jnp
from jax import lax
from jax.experimental import pallas as pl
from jax.experimental.pallas import tpu as pltpu
from jax.experimental.pallas import tpu_sc as plsc

_N = 10000
_E = 320000
_IN = 128
_H = 256
_OUT = 128

_NC = 2
_NS = 16
_NW = _NC * _NS
_C = 128
_NCH = 80
_EPT = _NCH * _C
_EPAD = _NW * _EPT
_SBCH = 4
_NSB = _NCH // _SBCH
_TSB = _NW * _NSB
_NSB0 = 28
_NSB1 = _NSB * 2 - _NSB0
_NA = 10112
_RPT = _NA // _NS
_ROW_CHUNKS = tuple((r0, min(_C, _RPT - r0)) for r0 in range(0, _RPT, _C))
_F = 128


def _sc_feat_body(table, src4, dst4, ewf, out_s,
                  src_v, dst_v, ew_v, rows0, rows1, acc, semA, semB):
  c = lax.axis_index("c")
  s = lax.axis_index("s")
  t0 = s * _RPT
  nsb = jnp.where(c == 0, _NSB0, _NSB1)
  sbbase = jnp.where(c == 0, s * _NSB0, _NS * _NSB0 + s * _NSB1)

  zeros16 = jnp.zeros((16,), jnp.float32)

  def zbody(r, carry):
    for d in range(_F // 16):
      rows0[r, pl.ds(d * 16, 16)] = zeros16
    return carry
  lax.fori_loop(0, _C, zbody, 0)

  for r0, rn in _ROW_CHUNKS:
    pltpu.sync_copy(rows0.at[pl.ds(0, rn)], acc.at[pl.ds(t0 + r0, rn)])
  plsc.subcore_barrier()

  rows = (rows0, rows1)
  sems = (semA, semB)

  def scale(rr, k):
    def body(e, cc):
      wv = ew_v[pl.ds((k * _C + e) * 16, 16)]
      for d in range(_F // 16):
        sl = pl.ds(d * 16, 16)
        rr[e, sl] = rr[e, sl] * wv
      return cc
    lax.fori_loop(0, _C, body, 0)

  def sb_body(b, carry):
    sbg = sbbase + b
    pltpu.sync_copy(src4.at[sbg], src_v)
    pltpu.sync_copy(dst4.at[sbg], dst_v)
    pltpu.sync_copy(ewf.at[pl.ds(sbg * (_SBCH * _C * 16), _SBCH * _C * 16)],
                    ew_v)
    pltpu.async_copy(table.at[src_v.at[0]], rows0, semA)
    for k in range(_SBCH):
      if k + 1 < _SBCH:
        pltpu.async_copy(table.at[src_v.at[k + 1]],
                         rows[(k + 1) % 2], sems[(k + 1) % 2])
      pltpu.make_async_copy(table.at[src_v.at[k]],
                            rows[k % 2], sems[k % 2]).wait()
      scale(rows[k % 2], k)
      pltpu.sync_copy(rows[k % 2], acc.at[dst_v.at[k]], add=True)
    return carry
  lax.fori_loop(0, nsb, sb_body, 0)
  plsc.subcore_barrier()

  for r0, rn in _ROW_CHUNKS:
    rb = t0 + r0
    pltpu.sync_copy(acc.at[pl.ds(rb, rn)], out_s.at[c, pl.ds(rb, rn)])


def _make_sc_feat():
  mesh = plsc.VectorSubcoreMesh(core_axis_name="c", subcore_axis_name="s")
  out_type = jax.ShapeDtypeStruct((_NC, _NA, _F), jnp.float32)
  scratch = [
      pltpu.VMEM((_SBCH, _C), jnp.int32),
      pltpu.VMEM((_SBCH, _C), jnp.int32),
      pltpu.VMEM((_SBCH * _C * 16,), jnp.float32),
      pltpu.VMEM((_C, _F), jnp.float32),
      pltpu.VMEM((_C, _F), jnp.float32),
      pltpu.VMEM_SHARED((_NA, _F), jnp.float32),
      pltpu.SemaphoreType.DMA,
      pltpu.SemaphoreType.DMA,
  ]
  return pl.kernel(_sc_feat_body, out_type=out_type, mesh=mesh,
                   scratch_types=scratch)


def _sc_deg_body(dst3, out_d, dst_v, ones_v, z16_v, dacc):
  c = lax.axis_index("c")
  s = lax.axis_index("s")
  wid = s * _NC + c
  t0 = s * _RPT

  zeros16 = jnp.zeros((16,), jnp.float32)
  ones16 = jnp.ones((16,), jnp.float32)

  def zbody(r, carry):
    ones_v[r, :] = ones16
    z16_v[r, :] = zeros16
    return carry
  lax.fori_loop(0, _C, zbody, 0)

  for r0, rn in _ROW_CHUNKS:
    pltpu.sync_copy(z16_v.at[pl.ds(0, rn)], dacc.at[pl.ds(t0 + r0, rn)])
  plsc.subcore_barrier()

  pltpu.sync_copy(dst3.at[wid], dst_v)

  def chunk(j, carry):
    pltpu.sync_copy(ones_v, dacc.at[dst_v.at[j]], add=True)
    return carry
  lax.fori_loop(0, _NCH, chunk, 0)
  plsc.subcore_barrier()

  for r0, rn in _ROW_CHUNKS:
    rb = t0 + r0
    pltpu.sync_copy(dacc.at[pl.ds(rb, rn)], out_d.at[c, pl.ds(rb, rn)])


def _make_sc_deg():
  mesh = plsc.VectorSubcoreMesh(core_axis_name="c", subcore_axis_name="s")
  out_type = jax.ShapeDtypeStruct((_NC, _NA, 16), jnp.float32)
  scratch = [
      pltpu.VMEM((_NCH, _C), jnp.int32),
      pltpu.VMEM((_C, 16), jnp.float32),
      pltpu.VMEM((_C, 16), jnp.float32),
      pltpu.VMEM_SHARED((_NA, 16), jnp.float32),
  ]
  return pl.kernel(_sc_deg_body, out_type=out_type, mesh=mesh,
                   scratch_types=scratch,
                   compiler_params=pltpu.CompilerParams(
                       use_tc_tiling_on_sc=False))


_sc_feat = _make_sc_feat()
_sc_degree = _make_sc_deg()


def _norm_body(w_ref, o_ref):
  w = w_ref[...]
  mn = jnp.min(w)
  mx = jnp.max(w)
  o_ref[...] = jnp.where(mx == mn, jnp.ones_like(w), (w - mn) / (mx - mn))


_norm = pl.pallas_call(
    _norm_body,
    out_shape=jax.ShapeDtypeStruct((_E // 128, 128), jnp.float32))


_BLK = 400
_NBLK = _N // _BLK


def _dense0_body(sp, dp, x, wl0, bl0, wr0, gamma, beta, rm, rv, wl1,
                 h_out, g_out):
  ssum = sp[0] + sp[1]
  dsum = dp[0] + dp[1]
  deg = jnp.clip(dsum[:, 0:1], 1.0, None)
  aggr = ssum / deg
  dn = (((1,), (1,)), ((), ()))
  pre = (lax.dot_general(aggr, wl0[...], dn, preferred_element_type=jnp.float32)
         + bl0[...]
         + lax.dot_general(x[...], wr0[...], dn, preferred_element_type=jnp.float32))
  inv = lax.rsqrt(rv[...] + 1e-5)
  hh = jnp.maximum((pre - rm[...]) * inv * gamma[...] + beta[...], 0.0)
  h_out[...] = hh
  g_out[...] = lax.dot_general(hh, wl1[...], dn, preferred_element_type=jnp.float32)


_dense0 = pl.pallas_call(
    _dense0_body,
    grid=(_NBLK,),
    in_specs=[
        pl.BlockSpec((_NC, _BLK, _F), lambda i: (0, i, 0)),
        pl.BlockSpec((_NC, _BLK, 16), lambda i: (0, i, 0)),
        pl.BlockSpec((_BLK, _IN), lambda i: (i, 0)),
        pl.BlockSpec((_H, _IN), lambda i: (0, 0)),
        pl.BlockSpec((1, _H), lambda i: (0, 0)),
        pl.BlockSpec((_H, _IN), lambda i: (0, 0)),
        pl.BlockSpec((1, _H), lambda i: (0, 0)),
        pl.BlockSpec((1, _H), lambda i: (0, 0)),
        pl.BlockSpec((1, _H), lambda i: (0, 0)),
        pl.BlockSpec((1, _H), lambda i: (0, 0)),
        pl.BlockSpec((_OUT, _H), lambda i: (0, 0)),
    ],
    out_specs=[
        pl.BlockSpec((_BLK, _H), lambda i: (i, 0)),
        pl.BlockSpec((_BLK, _OUT), lambda i: (i, 0)),
    ],
    out_shape=[
        jax.ShapeDtypeStruct((_N, _H), jnp.float32),
        jax.ShapeDtypeStruct((_N, _OUT), jnp.float32),
    ])


def _dense1_body(sp, dp, h, wr1, bl1, lsm_out, var_out, acc_s):
  i = pl.program_id(0)
  ssum = sp[0] + sp[1]
  dsum = dp[0] + dp[1]
  deg = jnp.clip(dsum[:, 0:1], 1.0, None)
  dn = (((1,), (1,)), ((), ()))
  o = (ssum / deg + bl1[...]
       + lax.dot_general(h[...], wr1[...], dn, preferred_element_type=jnp.float32))
  m = jnp.max(o, axis=1, keepdims=True)
  lse = jnp.log(jnp.sum(jnp.exp(o - m), axis=1, keepdims=True)) + m
  lsm_out[...] = o - lse
  bs = jnp.sum(o)
  bss = jnp.sum(o * o)

  @pl.when(i == 0)
  def _():
    acc_s[0] = bs
    acc_s[1] = bss

  @pl.when(i > 0)
  def _():
    acc_s[0] = acc_s[0] + bs
    acc_s[1] = acc_s[1] + bss

  tot = float(_N * _OUT)
  var_out[...] = jnp.full((1, 1), (acc_s[1] - acc_s[0] * acc_s[0] / tot)
                          / (tot - 1.0), jnp.float32)


_dense1 = pl.pallas_call(
    _dense1_body,
    grid=(_NBLK,),
    in_specs=[
        pl.BlockSpec((_NC, _BLK, _OUT), lambda i: (0, i, 0)),
        pl.BlockSpec((_NC, _BLK, 16), lambda i: (0, i, 0)),
        pl.BlockSpec((_BLK, _H), lambda i: (i, 0)),
        pl.BlockSpec((_OUT, _H), lambda i: (0, 0)),
        pl.BlockSpec((1, _OUT), lambda i: (0, 0)),
    ],
    out_specs=[
        pl.BlockSpec((_BLK, _OUT), lambda i: (i, 0)),
        pl.BlockSpec((1, 1), lambda i: (0, 0)),
    ],
    out_shape=[
        jax.ShapeDtypeStruct((_N, _OUT), jnp.float32),
        jax.ShapeDtypeStruct((1, 1), jnp.float32),
    ],
    scratch_shapes=[pltpu.SMEM((2,), jnp.float32)])


def kernel(x, edge_index, edge_weight, Wl0, bl0, Wr0, gamma, beta,
           running_mean, running_var, Wl1, bl1, Wr1):
  ewn = _norm(edge_weight.reshape(_E // 128, 128)).reshape(_E)
  pad = _EPAD - _E
  src4 = jnp.concatenate([edge_index[0], jnp.zeros((pad,), jnp.int32)]
                         ).reshape(_TSB, _SBCH, _C)
  junk = _N + jnp.arange(pad, dtype=jnp.int32) % (_NA - _N)
  dst_p = jnp.concatenate([edge_index[1], junk])
  dst3 = dst_p.reshape(_NW, _NCH, _C)
  dst4 = dst_p.reshape(_TSB, _SBCH, _C)
  ewn_p = jnp.concatenate([ewn, jnp.zeros((pad,), jnp.float32)])
  ewf = jnp.broadcast_to(ewn_p[:, None], (_EPAD, 16)).reshape(_EPAD * 16)

  dp0 = _sc_degree(dst3)
  sp0 = _sc_feat(x, src4, dst4, ewf)
  h, g = _dense0(sp0, dp0, x, Wl0, bl0.reshape(1, -1), Wr0,
                 gamma.reshape(1, -1), beta.reshape(1, -1),
                 running_mean.reshape(1, -1), running_var.reshape(1, -1), Wl1)
  sp1 = _sc_feat(g, src4, dst4, ewf)
  lsm, var = _dense1(sp1, dp0, h, Wr1, bl1.reshape(1, -1))
  return lsm, var.reshape(())


# speedup vs baseline: 1.1991x; 1.0486x over previous
import functools
import jax
import jax.numpy as jnp
from jax import lax
from jax.experimental import pallas as pl
from jax.experimental.pallas import tpu as pltpu
from jax.experimental.pallas import tpu_sc as plsc

_N = 10000
_E = 320000
_IN = 128
_H = 256
_OUT = 128

_NC = 2
_NS = 16
_NW = _NC * _NS
_C = 128
_NCH = 80
_EPT = _NCH * _C
_EPAD = _NW * _EPT
_SBCH = 4
_NSB = _NCH // _SBCH
_TSB = _NW * _NSB
_NSB0 = 30
_NSB1 = _NSB * 2 - _NSB0
_NA = 10112
_RPT = _NA // _NS
_ROW_CHUNKS = tuple((r0, min(_C, _RPT - r0)) for r0 in range(0, _RPT, _C))
_F = 128


def _sc_feat_body(table, src4, dst4, ewf, out_s,
                  src_v, dst_v, ew_v, rows0, rows1, acc, semA, semB):
  c = lax.axis_index("c")
  s = lax.axis_index("s")
  t0 = s * _RPT
  nsb = jnp.where(c == 0, _NSB0, _NSB1)
  sbbase = jnp.where(c == 0, s * _NSB0, _NS * _NSB0 + s * _NSB1)

  zeros16 = jnp.zeros((16,), jnp.float32)

  def zbody(r, carry):
    for d in range(_F // 16):
      rows0[r, pl.ds(d * 16, 16)] = zeros16
    return carry
  lax.fori_loop(0, _C, zbody, 0)

  for r0, rn in _ROW_CHUNKS:
    pltpu.sync_copy(rows0.at[pl.ds(0, rn)], acc.at[pl.ds(t0 + r0, rn)])
  plsc.subcore_barrier()

  rows = (rows0, rows1)
  sems = (semA, semB)

  def scale(rr, k):
    def body(e, cc):
      wv = ew_v[pl.ds((k * _C + e) * 16, 16)]
      for d in range(_F // 16):
        sl = pl.ds(d * 16, 16)
        rr[e, sl] = rr[e, sl] * wv
      return cc
    lax.fori_loop(0, _C, body, 0)

  def sb_body(b, carry):
    sbg = sbbase + b
    pltpu.sync_copy(src4.at[sbg], src_v)
    pltpu.sync_copy(dst4.at[sbg], dst_v)
    pltpu.sync_copy(ewf.at[pl.ds(sbg * (_SBCH * _C * 16), _SBCH * _C * 16)],
                    ew_v)
    pltpu.async_copy(table.at[src_v.at[0]], rows0, semA)
    for k in range(_SBCH):
      if k + 1 < _SBCH:
        pltpu.async_copy(table.at[src_v.at[k + 1]],
                         rows[(k + 1) % 2], sems[(k + 1) % 2])
      pltpu.make_async_copy(table.at[src_v.at[k]],
                            rows[k % 2], sems[k % 2]).wait()
      scale(rows[k % 2], k)
      pltpu.sync_copy(rows[k % 2], acc.at[dst_v.at[k]], add=True)
    return carry
  lax.fori_loop(0, nsb, sb_body, 0)
  plsc.subcore_barrier()

  for r0, rn in _ROW_CHUNKS:
    rb = t0 + r0
    pltpu.sync_copy(acc.at[pl.ds(rb, rn)], out_s.at[c, pl.ds(rb, rn)])


def _make_sc_feat():
  mesh = plsc.VectorSubcoreMesh(core_axis_name="c", subcore_axis_name="s")
  out_type = jax.ShapeDtypeStruct((_NC, _NA, _F), jnp.float32)
  scratch = [
      pltpu.VMEM((_SBCH, _C), jnp.int32),
      pltpu.VMEM((_SBCH, _C), jnp.int32),
      pltpu.VMEM((_SBCH * _C * 16,), jnp.float32),
      pltpu.VMEM((_C, _F), jnp.float32),
      pltpu.VMEM((_C, _F), jnp.float32),
      pltpu.VMEM_SHARED((_NA, _F), jnp.float32),
      pltpu.SemaphoreType.DMA,
      pltpu.SemaphoreType.DMA,
  ]
  return pl.kernel(_sc_feat_body, out_type=out_type, mesh=mesh,
                   scratch_types=scratch)


def _sc_deg_body(dst3, out_d, dst_v, ones_v, z16_v, dacc):
  c = lax.axis_index("c")
  s = lax.axis_index("s")
  wid = s * _NC + c
  t0 = s * _RPT

  zeros16 = jnp.zeros((16,), jnp.float32)
  ones16 = jnp.ones((16,), jnp.float32)

  def zbody(r, carry):
    ones_v[r, :] = ones16
    z16_v[r, :] = zeros16
    return carry
  lax.fori_loop(0, _C, zbody, 0)

  for r0, rn in _ROW_CHUNKS:
    pltpu.sync_copy(z16_v.at[pl.ds(0, rn)], dacc.at[pl.ds(t0 + r0, rn)])
  plsc.subcore_barrier()

  pltpu.sync_copy(dst3.at[wid], dst_v)

  def chunk(j, carry):
    pltpu.sync_copy(ones_v, dacc.at[dst_v.at[j]], add=True)
    return carry
  lax.fori_loop(0, _NCH, chunk, 0)
  plsc.subcore_barrier()

  for r0, rn in _ROW_CHUNKS:
    rb = t0 + r0
    pltpu.sync_copy(dacc.at[pl.ds(rb, rn)], out_d.at[c, pl.ds(rb, rn)])


def _make_sc_deg():
  mesh = plsc.VectorSubcoreMesh(core_axis_name="c", subcore_axis_name="s")
  out_type = jax.ShapeDtypeStruct((_NC, _NA, 16), jnp.float32)
  scratch = [
      pltpu.VMEM((_NCH, _C), jnp.int32),
      pltpu.VMEM((_C, 16), jnp.float32),
      pltpu.VMEM((_C, 16), jnp.float32),
      pltpu.VMEM_SHARED((_NA, 16), jnp.float32),
  ]
  return pl.kernel(_sc_deg_body, out_type=out_type, mesh=mesh,
                   scratch_types=scratch,
                   compiler_params=pltpu.CompilerParams(
                       use_tc_tiling_on_sc=False))


_sc_feat = _make_sc_feat()
_sc_degree = _make_sc_deg()


def _norm_body(w_ref, o_ref):
  w = w_ref[...]
  mn = jnp.min(w)
  mx = jnp.max(w)
  o_ref[...] = jnp.where(mx == mn, jnp.ones_like(w), (w - mn) / (mx - mn))


_norm = pl.pallas_call(
    _norm_body,
    out_shape=jax.ShapeDtypeStruct((_E // 128, 128), jnp.float32))


_BLK = 400
_NBLK = _N // _BLK


def _dense0_body(sp, dp, x, wl0, bl0, wr0, gamma, beta, rm, rv, wl1,
                 h_out, g_out):
  ssum = sp[0] + sp[1]
  dsum = dp[0] + dp[1]
  deg = jnp.clip(dsum[:, 0:1], 1.0, None)
  aggr = ssum / deg
  dn = (((1,), (1,)), ((), ()))
  pre = (lax.dot_general(aggr, wl0[...], dn, preferred_element_type=jnp.float32)
         + bl0[...]
         + lax.dot_general(x[...], wr0[...], dn, preferred_element_type=jnp.float32))
  inv = lax.rsqrt(rv[...] + 1e-5)
  hh = jnp.maximum((pre - rm[...]) * inv * gamma[...] + beta[...], 0.0)
  h_out[...] = hh
  g_out[...] = lax.dot_general(hh, wl1[...], dn, preferred_element_type=jnp.float32)


_dense0 = pl.pallas_call(
    _dense0_body,
    grid=(_NBLK,),
    in_specs=[
        pl.BlockSpec((_NC, _BLK, _F), lambda i: (0, i, 0)),
        pl.BlockSpec((_NC, _BLK, 16), lambda i: (0, i, 0)),
        pl.BlockSpec((_BLK, _IN), lambda i: (i, 0)),
        pl.BlockSpec((_H, _IN), lambda i: (0, 0)),
        pl.BlockSpec((1, _H), lambda i: (0, 0)),
        pl.BlockSpec((_H, _IN), lambda i: (0, 0)),
        pl.BlockSpec((1, _H), lambda i: (0, 0)),
        pl.BlockSpec((1, _H), lambda i: (0, 0)),
        pl.BlockSpec((1, _H), lambda i: (0, 0)),
        pl.BlockSpec((1, _H), lambda i: (0, 0)),
        pl.BlockSpec((_OUT, _H), lambda i: (0, 0)),
    ],
    out_specs=[
        pl.BlockSpec((_BLK, _H), lambda i: (i, 0)),
        pl.BlockSpec((_BLK, _OUT), lambda i: (i, 0)),
    ],
    out_shape=[
        jax.ShapeDtypeStruct((_N, _H), jnp.float32),
        jax.ShapeDtypeStruct((_N, _OUT), jnp.float32),
    ])


def _dense1_body(sp, dp, h, wr1, bl1, lsm_out, var_out, acc_s):
  i = pl.program_id(0)
  ssum = sp[0] + sp[1]
  dsum = dp[0] + dp[1]
  deg = jnp.clip(dsum[:, 0:1], 1.0, None)
  dn = (((1,), (1,)), ((), ()))
  o = (ssum / deg + bl1[...]
       + lax.dot_general(h[...], wr1[...], dn, preferred_element_type=jnp.float32))
  m = jnp.max(o, axis=1, keepdims=True)
  lse = jnp.log(jnp.sum(jnp.exp(o - m), axis=1, keepdims=True)) + m
  lsm_out[...] = o - lse
  bs = jnp.sum(o)
  bss = jnp.sum(o * o)

  @pl.when(i == 0)
  def _():
    acc_s[0] = bs
    acc_s[1] = bss

  @pl.when(i > 0)
  def _():
    acc_s[0] = acc_s[0] + bs
    acc_s[1] = acc_s[1] + bss

  tot = float(_N * _OUT)
  var_out[...] = jnp.full((1, 1), (acc_s[1] - acc_s[0] * acc_s[0] / tot)
                          / (tot - 1.0), jnp.float32)


_dense1 = pl.pallas_call(
    _dense1_body,
    grid=(_NBLK,),
    in_specs=[
        pl.BlockSpec((_NC, _BLK, _OUT), lambda i: (0, i, 0)),
        pl.BlockSpec((_NC, _BLK, 16), lambda i: (0, i, 0)),
        pl.BlockSpec((_BLK, _H), lambda i: (i, 0)),
        pl.BlockSpec((_OUT, _H), lambda i: (0, 0)),
        pl.BlockSpec((1, _OUT), lambda i: (0, 0)),
    ],
    out_specs=[
        pl.BlockSpec((_BLK, _OUT), lambda i: (i, 0)),
        pl.BlockSpec((1, 1), lambda i: (0, 0)),
    ],
    out_shape=[
        jax.ShapeDtypeStruct((_N, _OUT), jnp.float32),
        jax.ShapeDtypeStruct((1, 1), jnp.float32),
    ],
    scratch_shapes=[pltpu.SMEM((2,), jnp.float32)])


def kernel(x, edge_index, edge_weight, Wl0, bl0, Wr0, gamma, beta,
           running_mean, running_var, Wl1, bl1, Wr1):
  ewn = _norm(edge_weight.reshape(_E // 128, 128)).reshape(_E)
  pad = _EPAD - _E
  src4 = jnp.concatenate([edge_index[0], jnp.zeros((pad,), jnp.int32)]
                         ).reshape(_TSB, _SBCH, _C)
  junk = _N + jnp.arange(pad, dtype=jnp.int32) % (_NA - _N)
  dst_p = jnp.concatenate([edge_index[1], junk])
  dst3 = dst_p.reshape(_NW, _NCH, _C)
  dst4 = dst_p.reshape(_TSB, _SBCH, _C)
  ewn_p = jnp.concatenate([ewn, jnp.zeros((pad,), jnp.float32)])
  ewf = jnp.broadcast_to(ewn_p[:, None], (_EPAD, 16)).reshape(_EPAD * 16)

  dp0 = _sc_degree(dst3)
  sp0 = _sc_feat(x, src4, dst4, ewf)
  h, g = _dense0(sp0, dp0, x, Wl0, bl0.reshape(1, -1), Wr0,
                 gamma.reshape(1, -1), beta.reshape(1, -1),
                 running_mean.reshape(1, -1), running_var.reshape(1, -1), Wl1)
  sp1 = _sc_feat(g, src4, dst4, ewf)
  lsm, var = _dense1(sp1, dp0, h, Wr1, bl1.reshape(1, -1))
  return lsm, var.reshape(())


# speedup vs baseline: 1.3962x; 1.1644x over previous
import functools
import jax
import jax.numpy as jnp
from jax import lax
from jax.experimental import pallas as pl
from jax.experimental.pallas import tpu as pltpu
from jax.experimental.pallas import tpu_sc as plsc

_N = 10000
_E = 320000
_IN = 128
_H = 256
_OUT = 128

_NC = 2
_NS = 16
_NW = _NC * _NS
_C = 128
_NCH = 80
_EPT = _NCH * _C
_EPAD = _NW * _EPT
_SBCH = 4
_NSB = _NCH // _SBCH
_TSB = _NW * _NSB
_FH = 64
_NSB2 = _TSB // _NS
_NPT = _N // _NS
_NA = 10112
_RPT = _NA // _NS
_ROW_CHUNKS = tuple((r0, min(_C, _RPT - r0)) for r0 in range(0, _RPT, _C))
_F = 128


def _sc_feat_body(tabh, src4, dst4, ewf, out_s,
                  src_v, dst_v, ew_v, rows0, rows1, rows2, rows3,
                  tab_sp, acc, sem0, sem1, sem2, sem3):
  c = lax.axis_index("c")
  s = lax.axis_index("s")
  t0 = s * _RPT

  pltpu.sync_copy(tabh.at[c, pl.ds(s * _NPT, _NPT)],
                  tab_sp.at[pl.ds(s * _NPT, _NPT)])

  zeros16 = jnp.zeros((16,), jnp.float32)

  def zbody(r, carry):
    for d in range(_FH // 16):
      rows0[r, pl.ds(d * 16, 16)] = zeros16
    return carry
  lax.fori_loop(0, _C, zbody, 0)

  for r0, rn in _ROW_CHUNKS:
    pltpu.sync_copy(rows0.at[pl.ds(0, rn)], acc.at[pl.ds(t0 + r0, rn)])
  plsc.subcore_barrier()

  rows = (rows0, rows1, rows2, rows3)
  sems = (sem0, sem1, sem2, sem3)

  def scale(rr, k):
    def body(e, cc):
      wv = ew_v[pl.ds((k * _C + e) * 16, 16)]
      for d in range(_FH // 16):
        sl = pl.ds(d * 16, 16)
        rr[e, sl] = rr[e, sl] * wv
      return cc
    lax.fori_loop(0, _C, body, 0)

  def sb_body(b, carry):
    sbg = s * _NSB2 + b
    pltpu.sync_copy(src4.at[sbg], src_v)
    pltpu.sync_copy(dst4.at[sbg], dst_v)
    pltpu.sync_copy(ewf.at[pl.ds(sbg * (_SBCH * _C * 16), _SBCH * _C * 16)],
                    ew_v)
    for k in range(_SBCH):
      pltpu.async_copy(tab_sp.at[src_v.at[k]], rows[k], sems[k])
    for k in range(_SBCH):
      pltpu.make_async_copy(tab_sp.at[src_v.at[k]], rows[k], sems[k]).wait()
      scale(rows[k], k)
      pltpu.sync_copy(rows[k], acc.at[dst_v.at[k]], add=True)
    return carry
  lax.fori_loop(0, _NSB2, sb_body, 0)
  plsc.subcore_barrier()

  for r0, rn in _ROW_CHUNKS:
    rb = t0 + r0
    pltpu.sync_copy(acc.at[pl.ds(rb, rn)], out_s.at[c, pl.ds(rb, rn)])


def _make_sc_feat():
  mesh = plsc.VectorSubcoreMesh(core_axis_name="c", subcore_axis_name="s")
  out_type = jax.ShapeDtypeStruct((_NC, _NA, _FH), jnp.float32)
  scratch = [
      pltpu.VMEM((_SBCH, _C), jnp.int32),
      pltpu.VMEM((_SBCH, _C), jnp.int32),
      pltpu.VMEM((_SBCH * _C * 16,), jnp.float32),
      pltpu.VMEM((_C, _FH), jnp.float32),
      pltpu.VMEM((_C, _FH), jnp.float32),
      pltpu.VMEM((_C, _FH), jnp.float32),
      pltpu.VMEM((_C, _FH), jnp.float32),
      pltpu.VMEM_SHARED((_N, _FH), jnp.float32),
      pltpu.VMEM_SHARED((_NA, _FH), jnp.float32),
      pltpu.SemaphoreType.DMA,
      pltpu.SemaphoreType.DMA,
      pltpu.SemaphoreType.DMA,
      pltpu.SemaphoreType.DMA,
  ]
  return pl.kernel(_sc_feat_body, out_type=out_type, mesh=mesh,
                   scratch_types=scratch,
                   compiler_params=pltpu.CompilerParams(
                       use_tc_tiling_on_sc=False))


def _sc_deg_body(dst3, out_d, dst_v, ones_v, z16_v, dacc):
  c = lax.axis_index("c")
  s = lax.axis_index("s")
  wid = s * _NC + c
  t0 = s * _RPT

  zeros16 = jnp.zeros((16,), jnp.float32)
  ones16 = jnp.ones((16,), jnp.float32)

  def zbody(r, carry):
    ones_v[r, :] = ones16
    z16_v[r, :] = zeros16
    return carry
  lax.fori_loop(0, _C, zbody, 0)

  for r0, rn in _ROW_CHUNKS:
    pltpu.sync_copy(z16_v.at[pl.ds(0, rn)], dacc.at[pl.ds(t0 + r0, rn)])
  plsc.subcore_barrier()

  pltpu.sync_copy(dst3.at[wid], dst_v)

  def chunk(j, carry):
    pltpu.sync_copy(ones_v, dacc.at[dst_v.at[j]], add=True)
    return carry
  lax.fori_loop(0, _NCH, chunk, 0)
  plsc.subcore_barrier()

  for r0, rn in _ROW_CHUNKS:
    rb = t0 + r0
    pltpu.sync_copy(dacc.at[pl.ds(rb, rn)], out_d.at[c, pl.ds(rb, rn)])


def _make_sc_deg():
  mesh = plsc.VectorSubcoreMesh(core_axis_name="c", subcore_axis_name="s")
  out_type = jax.ShapeDtypeStruct((_NC, _NA, 16), jnp.float32)
  scratch = [
      pltpu.VMEM((_NCH, _C), jnp.int32),
      pltpu.VMEM((_C, 16), jnp.float32),
      pltpu.VMEM((_C, 16), jnp.float32),
      pltpu.VMEM_SHARED((_NA, 16), jnp.float32),
  ]
  return pl.kernel(_sc_deg_body, out_type=out_type, mesh=mesh,
                   scratch_types=scratch,
                   compiler_params=pltpu.CompilerParams(
                       use_tc_tiling_on_sc=False))


_sc_feat = _make_sc_feat()
_sc_degree = _make_sc_deg()


def _norm_body(w_ref, o_ref):
  w = w_ref[...]
  mn = jnp.min(w)
  mx = jnp.max(w)
  o_ref[...] = jnp.where(mx == mn, jnp.ones_like(w), (w - mn) / (mx - mn))


_norm = pl.pallas_call(
    _norm_body,
    out_shape=jax.ShapeDtypeStruct((_E // 128, 128), jnp.float32))


_BLK = 400
_NBLK = _N // _BLK


def _dense0_body(sp, dp, x, wl0, bl0, wr0, gamma, beta, rm, rv, wl1,
                 h_out, g_out):
  ssum = jnp.concatenate([sp[0], sp[1]], axis=1)
  dsum = dp[0] + dp[1]
  deg = jnp.clip(dsum[:, 0:1], 1.0, None)
  aggr = ssum / deg
  dn = (((1,), (1,)), ((), ()))
  pre = (lax.dot_general(aggr, wl0[...], dn, preferred_element_type=jnp.float32)
         + bl0[...]
         + lax.dot_general(x[...], wr0[...], dn, preferred_element_type=jnp.float32))
  inv = lax.rsqrt(rv[...] + 1e-5)
  hh = jnp.maximum((pre - rm[...]) * inv * gamma[...] + beta[...], 0.0)
  h_out[...] = hh
  g_out[...] = lax.dot_general(hh, wl1[...], dn, preferred_element_type=jnp.float32)


_dense0 = pl.pallas_call(
    _dense0_body,
    grid=(_NBLK,),
    in_specs=[
        pl.BlockSpec((_NC, _BLK, _FH), lambda i: (0, i, 0)),
        pl.BlockSpec((_NC, _BLK, 16), lambda i: (0, i, 0)),
        pl.BlockSpec((_BLK, _IN), lambda i: (i, 0)),
        pl.BlockSpec((_H, _IN), lambda i: (0, 0)),
        pl.BlockSpec((1, _H), lambda i: (0, 0)),
        pl.BlockSpec((_H, _IN), lambda i: (0, 0)),
        pl.BlockSpec((1, _H), lambda i: (0, 0)),
        pl.BlockSpec((1, _H), lambda i: (0, 0)),
        pl.BlockSpec((1, _H), lambda i: (0, 0)),
        pl.BlockSpec((1, _H), lambda i: (0, 0)),
        pl.BlockSpec((_OUT, _H), lambda i: (0, 0)),
    ],
    out_specs=[
        pl.BlockSpec((_BLK, _H), lambda i: (i, 0)),
        pl.BlockSpec((_BLK, _OUT), lambda i: (i, 0)),
    ],
    out_shape=[
        jax.ShapeDtypeStruct((_N, _H), jnp.float32),
        jax.ShapeDtypeStruct((_N, _OUT), jnp.float32),
    ])


def _dense1_body(sp, dp, h, wr1, bl1, lsm_out, var_out, acc_s):
  i = pl.program_id(0)
  ssum = jnp.concatenate([sp[0], sp[1]], axis=1)
  dsum = dp[0] + dp[1]
  deg = jnp.clip(dsum[:, 0:1], 1.0, None)
  dn = (((1,), (1,)), ((), ()))
  o = (ssum / deg + bl1[...]
       + lax.dot_general(h[...], wr1[...], dn, preferred_element_type=jnp.float32))
  m = jnp.max(o, axis=1, keepdims=True)
  lse = jnp.log(jnp.sum(jnp.exp(o - m), axis=1, keepdims=True)) + m
  lsm_out[...] = o - lse
  bs = jnp.sum(o)
  bss = jnp.sum(o * o)

  @pl.when(i == 0)
  def _():
    acc_s[0] = bs
    acc_s[1] = bss

  @pl.when(i > 0)
  def _():
    acc_s[0] = acc_s[0] + bs
    acc_s[1] = acc_s[1] + bss

  tot = float(_N * _OUT)
  var_out[...] = jnp.full((1, 1), (acc_s[1] - acc_s[0] * acc_s[0] / tot)
                          / (tot - 1.0), jnp.float32)


_dense1 = pl.pallas_call(
    _dense1_body,
    grid=(_NBLK,),
    in_specs=[
        pl.BlockSpec((_NC, _BLK, _FH), lambda i: (0, i, 0)),
        pl.BlockSpec((_NC, _BLK, 16), lambda i: (0, i, 0)),
        pl.BlockSpec((_BLK, _H), lambda i: (i, 0)),
        pl.BlockSpec((_OUT, _H), lambda i: (0, 0)),
        pl.BlockSpec((1, _OUT), lambda i: (0, 0)),
    ],
    out_specs=[
        pl.BlockSpec((_BLK, _OUT), lambda i: (i, 0)),
        pl.BlockSpec((1, 1), lambda i: (0, 0)),
    ],
    out_shape=[
        jax.ShapeDtypeStruct((_N, _OUT), jnp.float32),
        jax.ShapeDtypeStruct((1, 1), jnp.float32),
    ],
    scratch_shapes=[pltpu.SMEM((2,), jnp.float32)])


def kernel(x, edge_index, edge_weight, Wl0, bl0, Wr0, gamma, beta,
           running_mean, running_var, Wl1, bl1, Wr1):
  ewn = _norm(edge_weight.reshape(_E // 128, 128)).reshape(_E)
  pad = _EPAD - _E
  src4 = jnp.concatenate([edge_index[0], jnp.zeros((pad,), jnp.int32)]
                         ).reshape(_TSB, _SBCH, _C)
  junk = _N + jnp.arange(pad, dtype=jnp.int32) % (_NA - _N)
  dst_p = jnp.concatenate([edge_index[1], junk])
  dst3 = dst_p.reshape(_NW, _NCH, _C)
  dst4 = dst_p.reshape(_TSB, _SBCH, _C)
  ewn_p = jnp.concatenate([ewn, jnp.zeros((pad,), jnp.float32)])
  ewf = jnp.broadcast_to(ewn_p[:, None], (_EPAD, 16)).reshape(_EPAD * 16)

  dp0 = _sc_degree(dst3)
  xh = x.reshape(_N, 2, _FH).transpose(1, 0, 2)
  sp0 = _sc_feat(xh, src4, dst4, ewf)
  h, g = _dense0(sp0, dp0, x, Wl0, bl0.reshape(1, -1), Wr0,
                 gamma.reshape(1, -1), beta.reshape(1, -1),
                 running_mean.reshape(1, -1), running_var.reshape(1, -1), Wl1)
  gh = g.reshape(_N, 2, _FH).transpose(1, 0, 2)
  sp1 = _sc_feat(gh, src4, dst4, ewf)
  lsm, var = _dense1(sp1, dp0, h, Wr1, bl1.reshape(1, -1))
  return lsm, var.reshape(())


# speedup vs baseline: 1.5438x; 1.1057x over previous
import functools
import jax
import jax.numpy as jnp
from jax import lax
from jax.experimental import pallas as pl
from jax.experimental.pallas import tpu as pltpu
from jax.experimental.pallas import tpu_sc as plsc

_N = 10000
_E = 320000
_IN = 128
_H = 256
_OUT = 128

_NC = 2
_NS = 16
_NW = _NC * _NS
_C = 128
_NCH = 80
_EPT = _NCH * _C
_EPAD = _NW * _EPT
_SBCH = 4
_NSB = _NCH // _SBCH
_TSB = _NW * _NSB
_FH = 64
_NSB2 = _TSB // _NS
_NPT = _N // _NS
_NA = 10112
_RPT = _NA // _NS
_ROW_CHUNKS = tuple((r0, min(_C, _RPT - r0)) for r0 in range(0, _RPT, _C))
_F = 128


def _sc_feat_body(tabh, src4, dst4, ewf, out_s,
                  src_v, dst_v, ew_v, rows0, rows1, rows2, rows3,
                  tab_sp, acc, sem0, sem1, sem2, sem3,
                  ssem0, ssem1, ssem2, ssem3):
  c = lax.axis_index("c")
  s = lax.axis_index("s")
  t0 = s * _RPT

  pltpu.sync_copy(tabh.at[c, pl.ds(s * _NPT, _NPT)],
                  tab_sp.at[pl.ds(s * _NPT, _NPT)])

  zeros16 = jnp.zeros((16,), jnp.float32)

  def zbody(r, carry):
    for d in range(_FH // 16):
      rows0[r, pl.ds(d * 16, 16)] = zeros16
    return carry
  lax.fori_loop(0, _C, zbody, 0)

  for r0, rn in _ROW_CHUNKS:
    pltpu.sync_copy(rows0.at[pl.ds(0, rn)], acc.at[pl.ds(t0 + r0, rn)])
  plsc.subcore_barrier()

  rows = (rows0, rows1, rows2, rows3)
  sems = (sem0, sem1, sem2, sem3)
  ssems = (ssem0, ssem1, ssem2, ssem3)

  def scale(rr, k):
    def body(e, cc):
      wv = ew_v[pl.ds((k * _C + e) * 16, 16)]
      for d in range(_FH // 16):
        sl = pl.ds(d * 16, 16)
        rr[e, sl] = rr[e, sl] * wv
      return cc
    lax.fori_loop(0, _C, body, 0)

  def sb_body(b, carry):
    sbg = s * _NSB2 + b
    par = lax.rem(b, 2) * _SBCH
    pltpu.sync_copy(src4.at[sbg], src_v)
    pltpu.sync_copy(dst4.at[sbg], dst_v.at[pl.ds(par, _SBCH)])
    pltpu.sync_copy(ewf.at[pl.ds(sbg * (_SBCH * _C * 16), _SBCH * _C * 16)],
                    ew_v)
    for k in range(_SBCH):
      @pl.when(b > 0)
      def _():
        pltpu.make_async_copy(rows[k], acc.at[dst_v.at[par + k]],
                              ssems[k]).wait()
      pltpu.async_copy(tab_sp.at[src_v.at[k]], rows[k], sems[k])
    for k in range(_SBCH):
      pltpu.make_async_copy(tab_sp.at[src_v.at[k]], rows[k], sems[k]).wait()
      scale(rows[k], k)
      pltpu.async_copy(rows[k], acc.at[dst_v.at[par + k]], ssems[k],
                       add=True)
    return carry
  lax.fori_loop(0, _NSB2, sb_body, 0)
  lastp = ((_NSB2 - 1) % 2) * _SBCH
  for k in range(_SBCH):
    pltpu.make_async_copy(rows[k], acc.at[dst_v.at[lastp + k]],
                          ssems[k]).wait()
  plsc.subcore_barrier()

  for r0, rn in _ROW_CHUNKS:
    rb = t0 + r0
    pltpu.sync_copy(acc.at[pl.ds(rb, rn)], out_s.at[c, pl.ds(rb, rn)])


def _make_sc_feat():
  mesh = plsc.VectorSubcoreMesh(core_axis_name="c", subcore_axis_name="s")
  out_type = jax.ShapeDtypeStruct((_NC, _NA, _FH), jnp.float32)
  scratch = [
      pltpu.VMEM((_SBCH, _C), jnp.int32),
      pltpu.VMEM((2 * _SBCH, _C), jnp.int32),
      pltpu.VMEM((_SBCH * _C * 16,), jnp.float32),
      pltpu.VMEM((_C, _FH), jnp.float32),
      pltpu.VMEM((_C, _FH), jnp.float32),
      pltpu.VMEM((_C, _FH), jnp.float32),
      pltpu.VMEM((_C, _FH), jnp.float32),
      pltpu.VMEM_SHARED((_N, _FH), jnp.float32),
      pltpu.VMEM_SHARED((_NA, _FH), jnp.float32),
      pltpu.SemaphoreType.DMA,
      pltpu.SemaphoreType.DMA,
      pltpu.SemaphoreType.DMA,
      pltpu.SemaphoreType.DMA,
      pltpu.SemaphoreType.DMA,
      pltpu.SemaphoreType.DMA,
      pltpu.SemaphoreType.DMA,
      pltpu.SemaphoreType.DMA,
  ]
  return pl.kernel(_sc_feat_body, out_type=out_type, mesh=mesh,
                   scratch_types=scratch,
                   compiler_params=pltpu.CompilerParams(
                       use_tc_tiling_on_sc=False))


def _sc_deg_body(dst3, out_d, dst_v, ones_v, z16_v, dacc):
  c = lax.axis_index("c")
  s = lax.axis_index("s")
  wid = s * _NC + c
  t0 = s * _RPT

  zeros16 = jnp.zeros((16,), jnp.float32)
  ones16 = jnp.ones((16,), jnp.float32)

  def zbody(r, carry):
    ones_v[r, :] = ones16
    z16_v[r, :] = zeros16
    return carry
  lax.fori_loop(0, _C, zbody, 0)

  for r0, rn in _ROW_CHUNKS:
    pltpu.sync_copy(z16_v.at[pl.ds(0, rn)], dacc.at[pl.ds(t0 + r0, rn)])
  plsc.subcore_barrier()

  pltpu.sync_copy(dst3.at[wid], dst_v)

  def chunk(j, carry):
    pltpu.sync_copy(ones_v, dacc.at[dst_v.at[j]], add=True)
    return carry
  lax.fori_loop(0, _NCH, chunk, 0)
  plsc.subcore_barrier()

  for r0, rn in _ROW_CHUNKS:
    rb = t0 + r0
    pltpu.sync_copy(dacc.at[pl.ds(rb, rn)], out_d.at[c, pl.ds(rb, rn)])


def _make_sc_deg():
  mesh = plsc.VectorSubcoreMesh(core_axis_name="c", subcore_axis_name="s")
  out_type = jax.ShapeDtypeStruct((_NC, _NA, 16), jnp.float32)
  scratch = [
      pltpu.VMEM((_NCH, _C), jnp.int32),
      pltpu.VMEM((_C, 16), jnp.float32),
      pltpu.VMEM((_C, 16), jnp.float32),
      pltpu.VMEM_SHARED((_NA, 16), jnp.float32),
  ]
  return pl.kernel(_sc_deg_body, out_type=out_type, mesh=mesh,
                   scratch_types=scratch,
                   compiler_params=pltpu.CompilerParams(
                       use_tc_tiling_on_sc=False))


_sc_feat = _make_sc_feat()
_sc_degree = _make_sc_deg()


def _norm_body(w_ref, o_ref):
  w = w_ref[...]
  mn = jnp.min(w)
  mx = jnp.max(w)
  o_ref[...] = jnp.where(mx == mn, jnp.ones_like(w), (w - mn) / (mx - mn))


_norm = pl.pallas_call(
    _norm_body,
    out_shape=jax.ShapeDtypeStruct((_E // 128, 128), jnp.float32))


_BLK = 400
_NBLK = _N // _BLK


def _dense0_body(sp, dp, x, wl0, bl0, wr0, gamma, beta, rm, rv, wl1,
                 h_out, g_out):
  ssum = jnp.concatenate([sp[0], sp[1]], axis=1)
  dsum = dp[0] + dp[1]
  deg = jnp.clip(dsum[:, 0:1], 1.0, None)
  aggr = ssum / deg
  dn = (((1,), (1,)), ((), ()))
  pre = (lax.dot_general(aggr, wl0[...], dn, preferred_element_type=jnp.float32)
         + bl0[...]
         + lax.dot_general(x[...], wr0[...], dn, preferred_element_type=jnp.float32))
  inv = lax.rsqrt(rv[...] + 1e-5)
  hh = jnp.maximum((pre - rm[...]) * inv * gamma[...] + beta[...], 0.0)
  h_out[...] = hh
  g_out[...] = lax.dot_general(hh, wl1[...], dn, preferred_element_type=jnp.float32)


_dense0 = pl.pallas_call(
    _dense0_body,
    grid=(_NBLK,),
    in_specs=[
        pl.BlockSpec((_NC, _BLK, _FH), lambda i: (0, i, 0)),
        pl.BlockSpec((_NC, _BLK, 16), lambda i: (0, i, 0)),
        pl.BlockSpec((_BLK, _IN), lambda i: (i, 0)),
        pl.BlockSpec((_H, _IN), lambda i: (0, 0)),
        pl.BlockSpec((1, _H), lambda i: (0, 0)),
        pl.BlockSpec((_H, _IN), lambda i: (0, 0)),
        pl.BlockSpec((1, _H), lambda i: (0, 0)),
        pl.BlockSpec((1, _H), lambda i: (0, 0)),
        pl.BlockSpec((1, _H), lambda i: (0, 0)),
        pl.BlockSpec((1, _H), lambda i: (0, 0)),
        pl.BlockSpec((_OUT, _H), lambda i: (0, 0)),
    ],
    out_specs=[
        pl.BlockSpec((_BLK, _H), lambda i: (i, 0)),
        pl.BlockSpec((_BLK, _OUT), lambda i: (i, 0)),
    ],
    out_shape=[
        jax.ShapeDtypeStruct((_N, _H), jnp.float32),
        jax.ShapeDtypeStruct((_N, _OUT), jnp.float32),
    ])


def _dense1_body(sp, dp, h, wr1, bl1, lsm_out, var_out, acc_s):
  i = pl.program_id(0)
  ssum = jnp.concatenate([sp[0], sp[1]], axis=1)
  dsum = dp[0] + dp[1]
  deg = jnp.clip(dsum[:, 0:1], 1.0, None)
  dn = (((1,), (1,)), ((), ()))
  o = (ssum / deg + bl1[...]
       + lax.dot_general(h[...], wr1[...], dn, preferred_element_type=jnp.float32))
  m = jnp.max(o, axis=1, keepdims=True)
  lse = jnp.log(jnp.sum(jnp.exp(o - m), axis=1, keepdims=True)) + m
  lsm_out[...] = o - lse
  bs = jnp.sum(o)
  bss = jnp.sum(o * o)

  @pl.when(i == 0)
  def _():
    acc_s[0] = bs
    acc_s[1] = bss

  @pl.when(i > 0)
  def _():
    acc_s[0] = acc_s[0] + bs
    acc_s[1] = acc_s[1] + bss

  tot = float(_N * _OUT)
  var_out[...] = jnp.full((1, 1), (acc_s[1] - acc_s[0] * acc_s[0] / tot)
                          / (tot - 1.0), jnp.float32)


_dense1 = pl.pallas_call(
    _dense1_body,
    grid=(_NBLK,),
    in_specs=[
        pl.BlockSpec((_NC, _BLK, _FH), lambda i: (0, i, 0)),
        pl.BlockSpec((_NC, _BLK, 16), lambda i: (0, i, 0)),
        pl.BlockSpec((_BLK, _H), lambda i: (i, 0)),
        pl.BlockSpec((_OUT, _H), lambda i: (0, 0)),
        pl.BlockSpec((1, _OUT), lambda i: (0, 0)),
    ],
    out_specs=[
        pl.BlockSpec((_BLK, _OUT), lambda i: (i, 0)),
        pl.BlockSpec((1, 1), lambda i: (0, 0)),
    ],
    out_shape=[
        jax.ShapeDtypeStruct((_N, _OUT), jnp.float32),
        jax.ShapeDtypeStruct((1, 1), jnp.float32),
    ],
    scratch_shapes=[pltpu.SMEM((2,), jnp.float32)])


def kernel(x, edge_index, edge_weight, Wl0, bl0, Wr0, gamma, beta,
           running_mean, running_var, Wl1, bl1, Wr1):
  ewn = _norm(edge_weight.reshape(_E // 128, 128)).reshape(_E)
  pad = _EPAD - _E
  src4 = jnp.concatenate([edge_index[0], jnp.zeros((pad,), jnp.int32)]
                         ).reshape(_TSB, _SBCH, _C)
  junk = _N + jnp.arange(pad, dtype=jnp.int32) % (_NA - _N)
  dst_p = jnp.concatenate([edge_index[1], junk])
  dst3 = dst_p.reshape(_NW, _NCH, _C)
  dst4 = dst_p.reshape(_TSB, _SBCH, _C)
  ewn_p = jnp.concatenate([ewn, jnp.zeros((pad,), jnp.float32)])
  ewf = jnp.broadcast_to(ewn_p[:, None], (_EPAD, 16)).reshape(_EPAD * 16)

  dp0 = _sc_degree(dst3)
  xh = x.reshape(_N, 2, _FH).transpose(1, 0, 2)
  sp0 = _sc_feat(xh, src4, dst4, ewf)
  h, g = _dense0(sp0, dp0, x, Wl0, bl0.reshape(1, -1), Wr0,
                 gamma.reshape(1, -1), beta.reshape(1, -1),
                 running_mean.reshape(1, -1), running_var.reshape(1, -1), Wl1)
  gh = g.reshape(_N, 2, _FH).transpose(1, 0, 2)
  sp1 = _sc_feat(gh, src4, dst4, ewf)
  lsm, var = _dense1(sp1, dp0, h, Wr1, bl1.reshape(1, -1))
  return lsm, var.reshape(())


# speedup vs baseline: 2.2852x; 1.4802x over previous
import functools
import jax
import jax.numpy as jnp
from jax import lax
from jax.experimental import pallas as pl
from jax.experimental.pallas import tpu as pltpu
from jax.experimental.pallas import tpu_sc as plsc

_N = 10000
_E = 320000
_IN = 128
_H = 256
_OUT = 128

_NC = 2
_NS = 16
_NW = _NC * _NS
_C = 128
_NCH = 80
_EPT = _NCH * _C
_EPAD = _NW * _EPT
_SBCH = 4
_NSB = _NCH // _SBCH
_TSB = _NW * _NSB
_FH = 64
_NSB2 = _TSB // _NS
_NPT = _N // _NS
_NA = 10112
_RPT = _NA // _NS
_ROW_CHUNKS = tuple((r0, min(_C, _RPT - r0)) for r0 in range(0, _RPT, _C))
_F = 128


def _sc_feat_body(tabh, src4, dst4, ewf, out_s,
                  src_v, dst_v, ew_v, rows0, rows1, rows2, rows3,
                  tab_sp, acc, sem0, sem1, sem2, sem3,
                  ssem0, ssem1, ssem2, ssem3):
  c = lax.axis_index("c")
  s = lax.axis_index("s")
  t0 = s * _RPT

  pltpu.sync_copy(tabh.at[c, pl.ds(s * _NPT, _NPT)],
                  tab_sp.at[pl.ds(s * _NPT, _NPT)])

  zeros16 = jnp.zeros((16,), jnp.float32)

  def zbody(r, carry):
    for d in range(_FH // 16):
      rows0[r, pl.ds(d * 16, 16)] = zeros16
    return carry
  lax.fori_loop(0, _C, zbody, 0)

  for r0, rn in _ROW_CHUNKS:
    pltpu.sync_copy(rows0.at[pl.ds(0, rn)], acc.at[pl.ds(t0 + r0, rn)])
  plsc.subcore_barrier()

  rows = (rows0, rows1, rows2, rows3)
  sems = (sem0, sem1, sem2, sem3)
  ssems = (ssem0, ssem1, ssem2, ssem3)

  def scale(rr, k):
    def body(e, cc):
      wv = ew_v[e, pl.ds(k * 16, 16)]
      for d in range(_FH // 16):
        sl = pl.ds(d * 16, 16)
        rr[e, sl] = rr[e, sl] * wv
      return cc
    lax.fori_loop(0, _C, body, 0)

  def sb_body(b, carry):
    sbg = s * _NSB2 + b
    par = lax.rem(b, 2) * _SBCH
    pltpu.sync_copy(src4.at[sbg], src_v)
    pltpu.sync_copy(dst4.at[sbg], dst_v.at[pl.ds(par, _SBCH)])
    pltpu.sync_copy(ewf.at[:, pl.ds(sbg * 64, 64)], ew_v)
    for k in range(_SBCH):
      @pl.when(b > 0)
      def _():
        pltpu.make_async_copy(rows[k], acc.at[dst_v.at[par + k]],
                              ssems[k]).wait()
      pltpu.async_copy(tab_sp.at[src_v.at[k]], rows[k], sems[k])
    for k in range(_SBCH):
      pltpu.make_async_copy(tab_sp.at[src_v.at[k]], rows[k], sems[k]).wait()
      scale(rows[k], k)
      pltpu.async_copy(rows[k], acc.at[dst_v.at[par + k]], ssems[k],
                       add=True)
    return carry
  lax.fori_loop(0, _NSB2, sb_body, 0)
  lastp = ((_NSB2 - 1) % 2) * _SBCH
  for k in range(_SBCH):
    pltpu.make_async_copy(rows[k], acc.at[dst_v.at[lastp + k]],
                          ssems[k]).wait()
  plsc.subcore_barrier()

  for r0, rn in _ROW_CHUNKS:
    rb = t0 + r0
    pltpu.sync_copy(acc.at[pl.ds(rb, rn)], out_s.at[c, pl.ds(rb, rn)])


def _make_sc_feat():
  mesh = plsc.VectorSubcoreMesh(core_axis_name="c", subcore_axis_name="s")
  out_type = jax.ShapeDtypeStruct((_NC, _NA, _FH), jnp.float32)
  scratch = [
      pltpu.VMEM((_SBCH, _C), jnp.int32),
      pltpu.VMEM((2 * _SBCH, _C), jnp.int32),
      pltpu.VMEM((128, 64), jnp.float32),
      pltpu.VMEM((_C, _FH), jnp.float32),
      pltpu.VMEM((_C, _FH), jnp.float32),
      pltpu.VMEM((_C, _FH), jnp.float32),
      pltpu.VMEM((_C, _FH), jnp.float32),
      pltpu.VMEM_SHARED((_N, _FH), jnp.float32),
      pltpu.VMEM_SHARED((_NA, _FH), jnp.float32),
      pltpu.SemaphoreType.DMA,
      pltpu.SemaphoreType.DMA,
      pltpu.SemaphoreType.DMA,
      pltpu.SemaphoreType.DMA,
      pltpu.SemaphoreType.DMA,
      pltpu.SemaphoreType.DMA,
      pltpu.SemaphoreType.DMA,
      pltpu.SemaphoreType.DMA,
  ]
  return pl.kernel(_sc_feat_body, out_type=out_type, mesh=mesh,
                   scratch_types=scratch,
                   compiler_params=pltpu.CompilerParams(
                       use_tc_tiling_on_sc=False))


def _sc_deg_body(dst3, out_d, dst_v, ones_v, z16_v, dacc):
  c = lax.axis_index("c")
  s = lax.axis_index("s")
  wid = s * _NC + c
  t0 = s * _RPT

  zeros16 = jnp.zeros((16,), jnp.float32)
  ones16 = jnp.ones((16,), jnp.float32)

  def zbody(r, carry):
    ones_v[r, :] = ones16
    z16_v[r, :] = zeros16
    return carry
  lax.fori_loop(0, _C, zbody, 0)

  for r0, rn in _ROW_CHUNKS:
    pltpu.sync_copy(z16_v.at[pl.ds(0, rn)], dacc.at[pl.ds(t0 + r0, rn)])
  plsc.subcore_barrier()

  pltpu.sync_copy(dst3.at[wid], dst_v)

  def chunk(j, carry):
    pltpu.sync_copy(ones_v, dacc.at[dst_v.at[j]], add=True)
    return carry
  lax.fori_loop(0, _NCH, chunk, 0)
  plsc.subcore_barrier()

  for r0, rn in _ROW_CHUNKS:
    rb = t0 + r0
    pltpu.sync_copy(dacc.at[pl.ds(rb, rn)], out_d.at[c, pl.ds(rb, rn)])


def _make_sc_deg():
  mesh = plsc.VectorSubcoreMesh(core_axis_name="c", subcore_axis_name="s")
  out_type = jax.ShapeDtypeStruct((_NC, _NA, 16), jnp.float32)
  scratch = [
      pltpu.VMEM((_NCH, _C), jnp.int32),
      pltpu.VMEM((_C, 16), jnp.float32),
      pltpu.VMEM((_C, 16), jnp.float32),
      pltpu.VMEM_SHARED((_NA, 16), jnp.float32),
  ]
  return pl.kernel(_sc_deg_body, out_type=out_type, mesh=mesh,
                   scratch_types=scratch,
                   compiler_params=pltpu.CompilerParams(
                       use_tc_tiling_on_sc=False))


_sc_feat = _make_sc_feat()
_sc_degree = _make_sc_deg()


def _minmax_body(w_ref, mn_ref, mx_ref):
  w = w_ref[...]
  mn_ref[...] = jnp.full((1, 1), jnp.min(w), jnp.float32)
  mx_ref[...] = jnp.full((1, 1), jnp.max(w), jnp.float32)


_minmax = pl.pallas_call(
    _minmax_body,
    out_shape=[jax.ShapeDtypeStruct((1, 1), jnp.float32),
               jax.ShapeDtypeStruct((1, 1), jnp.float32)])

_RB = 512
_RG = (_EPAD * 16 // 128) // _RB


def _rep_body(mn_ref, mx_ref, w_ref, o_ref):
  i = pl.program_id(0)
  mn = mn_ref[0, 0]
  mx = mx_ref[0, 0]
  eq = mx == mn
  a = jnp.where(eq, 0.0, 1.0 / jnp.where(eq, 1.0, mx - mn))
  b = jnp.where(eq, 1.0, -mn * a)
  w = w_ref[...]
  rr = lax.broadcasted_iota(jnp.int32, (32, _RB), 0)
  rc = lax.broadcasted_iota(jnp.int32, (32, _RB), 1)
  R = (rc // 16 == rr).astype(jnp.float32)
  rep = lax.dot_general(w, R, (((0,), (0,)), ((), ())),
                        preferred_element_type=jnp.float32)
  r_io = lax.broadcasted_iota(jnp.int32, (128, _RB), 0)
  c_io = lax.broadcasted_iota(jnp.int32, (128, _RB), 1)
  e = 4096 * i + 128 * (c_io // 16) + r_io
  o_ref[...] = jnp.where(e < _E, rep * a + b, 0.0)


_rep = pl.pallas_call(
    _rep_body,
    grid=(_RG,),
    in_specs=[
        pl.BlockSpec((1, 1), lambda i: (0, 0)),
        pl.BlockSpec((1, 1), lambda i: (0, 0)),
        pl.BlockSpec((32, 128), lambda i: (i, 0)),
    ],
    out_specs=pl.BlockSpec((128, _RB), lambda i: (0, i)),
    out_shape=jax.ShapeDtypeStruct((128, _EPAD * 16 // 128), jnp.float32))


_BLK = 400
_NBLK = _N // _BLK


def _dense0_body(sp, dp, x, wl0, bl0, wr0, gamma, beta, rm, rv, wl1,
                 h_out, g_out):
  ssum = jnp.concatenate([sp[0], sp[1]], axis=1)
  dsum = dp[0] + dp[1]
  deg = jnp.clip(dsum[:, 0:1], 1.0, None)
  aggr = ssum / deg
  dn = (((1,), (1,)), ((), ()))
  pre = (lax.dot_general(aggr, wl0[...], dn, preferred_element_type=jnp.float32)
         + bl0[...]
         + lax.dot_general(x[...], wr0[...], dn, preferred_element_type=jnp.float32))
  inv = lax.rsqrt(rv[...] + 1e-5)
  hh = jnp.maximum((pre - rm[...]) * inv * gamma[...] + beta[...], 0.0)
  h_out[...] = hh
  g_out[...] = lax.dot_general(hh, wl1[...], dn, preferred_element_type=jnp.float32)


_dense0 = pl.pallas_call(
    _dense0_body,
    grid=(_NBLK,),
    in_specs=[
        pl.BlockSpec((_NC, _BLK, _FH), lambda i: (0, i, 0)),
        pl.BlockSpec((_NC, _BLK, 16), lambda i: (0, i, 0)),
        pl.BlockSpec((_BLK, _IN), lambda i: (i, 0)),
        pl.BlockSpec((_H, _IN), lambda i: (0, 0)),
        pl.BlockSpec((1, _H), lambda i: (0, 0)),
        pl.BlockSpec((_H, _IN), lambda i: (0, 0)),
        pl.BlockSpec((1, _H), lambda i: (0, 0)),
        pl.BlockSpec((1, _H), lambda i: (0, 0)),
        pl.BlockSpec((1, _H), lambda i: (0, 0)),
        pl.BlockSpec((1, _H), lambda i: (0, 0)),
        pl.BlockSpec((_OUT, _H), lambda i: (0, 0)),
    ],
    out_specs=[
        pl.BlockSpec((_BLK, _H), lambda i: (i, 0)),
        pl.BlockSpec((_BLK, _OUT), lambda i: (i, 0)),
    ],
    out_shape=[
        jax.ShapeDtypeStruct((_N, _H), jnp.float32),
        jax.ShapeDtypeStruct((_N, _OUT), jnp.float32),
    ])


def _dense1_body(sp, dp, h, wr1, bl1, lsm_out, var_out, acc_s):
  i = pl.program_id(0)
  ssum = jnp.concatenate([sp[0], sp[1]], axis=1)
  dsum = dp[0] + dp[1]
  deg = jnp.clip(dsum[:, 0:1], 1.0, None)
  dn = (((1,), (1,)), ((), ()))
  o = (ssum / deg + bl1[...]
       + lax.dot_general(h[...], wr1[...], dn, preferred_element_type=jnp.float32))
  m = jnp.max(o, axis=1, keepdims=True)
  lse = jnp.log(jnp.sum(jnp.exp(o - m), axis=1, keepdims=True)) + m
  lsm_out[...] = o - lse
  bs = jnp.sum(o)
  bss = jnp.sum(o * o)

  @pl.when(i == 0)
  def _():
    acc_s[0] = bs
    acc_s[1] = bss

  @pl.when(i > 0)
  def _():
    acc_s[0] = acc_s[0] + bs
    acc_s[1] = acc_s[1] + bss

  tot = float(_N * _OUT)
  var_out[...] = jnp.full((1, 1), (acc_s[1] - acc_s[0] * acc_s[0] / tot)
                          / (tot - 1.0), jnp.float32)


_dense1 = pl.pallas_call(
    _dense1_body,
    grid=(_NBLK,),
    in_specs=[
        pl.BlockSpec((_NC, _BLK, _FH), lambda i: (0, i, 0)),
        pl.BlockSpec((_NC, _BLK, 16), lambda i: (0, i, 0)),
        pl.BlockSpec((_BLK, _H), lambda i: (i, 0)),
        pl.BlockSpec((_OUT, _H), lambda i: (0, 0)),
        pl.BlockSpec((1, _OUT), lambda i: (0, 0)),
    ],
    out_specs=[
        pl.BlockSpec((_BLK, _OUT), lambda i: (i, 0)),
        pl.BlockSpec((1, 1), lambda i: (0, 0)),
    ],
    out_shape=[
        jax.ShapeDtypeStruct((_N, _OUT), jnp.float32),
        jax.ShapeDtypeStruct((1, 1), jnp.float32),
    ],
    scratch_shapes=[pltpu.SMEM((2,), jnp.float32)])


def kernel(x, edge_index, edge_weight, Wl0, bl0, Wr0, gamma, beta,
           running_mean, running_var, Wl1, bl1, Wr1):
  pad = _EPAD - _E
  mn, mx = _minmax(edge_weight.reshape(_E // 128, 128))
  wraw = jnp.concatenate([edge_weight, jnp.zeros((pad,), jnp.float32)]
                         ).reshape(_EPAD // 128, 128)
  ewf = _rep(mn, mx, wraw)
  src4 = jnp.concatenate([edge_index[0], jnp.zeros((pad,), jnp.int32)]
                         ).reshape(_TSB, _SBCH, _C)
  junk = _N + jnp.arange(pad, dtype=jnp.int32) % (_NA - _N)
  dst_p = jnp.concatenate([edge_index[1], junk])
  dst3 = dst_p.reshape(_NW, _NCH, _C)
  dst4 = dst_p.reshape(_TSB, _SBCH, _C)

  dp0 = _sc_degree(dst3)
  xh = x.reshape(_N, 2, _FH).transpose(1, 0, 2)
  sp0 = _sc_feat(xh, src4, dst4, ewf)
  h, g = _dense0(sp0, dp0, x, Wl0, bl0.reshape(1, -1), Wr0,
                 gamma.reshape(1, -1), beta.reshape(1, -1),
                 running_mean.reshape(1, -1), running_var.reshape(1, -1), Wl1)
  gh = g.reshape(_N, 2, _FH).transpose(1, 0, 2)
  sp1 = _sc_feat(gh, src4, dst4, ewf)
  lsm, var = _dense1(sp1, dp0, h, Wr1, bl1.reshape(1, -1))
  return lsm, var.reshape(())


# speedup vs baseline: 2.4004x; 1.0504x over previous
import functools
import jax
import jax.numpy as jnp
from jax import lax
from jax.experimental import pallas as pl
from jax.experimental.pallas import tpu as pltpu
from jax.experimental.pallas import tpu_sc as plsc

_N = 10000
_E = 320000
_IN = 128
_H = 256
_OUT = 128

_NC = 2
_NS = 16
_NW = _NC * _NS
_C = 128
_NCH = 80
_EPT = _NCH * _C
_EPAD = _NW * _EPT
_SBCH = 4
_NSB = _NCH // _SBCH
_TSB = _NW * _NSB
_FH = 64
_NSB2 = _TSB // _NS
_NPT = _N // _NS
_NA = 10112
_RPT = _NA // _NS
_ROW_CHUNKS = tuple((r0, min(_C, _RPT - r0)) for r0 in range(0, _RPT, _C))
_F = 128


def _sc_feat_body(tabh, src4, dst4, ewf, out_s,
                  src_v, dst_v, ew_v, rows0, rows1, rows2, rows3,
                  tab_sp, acc, sem0, sem1, sem2, sem3,
                  ssem0, ssem1, ssem2, ssem3):
  c = lax.axis_index("c")
  s = lax.axis_index("s")
  t0 = s * _RPT

  pltpu.sync_copy(tabh.at[c, pl.ds(s * _NPT, _NPT)],
                  tab_sp.at[pl.ds(s * _NPT, _NPT)])

  zeros16 = jnp.zeros((16,), jnp.float32)

  def zbody(r, carry):
    for d in range(_FH // 16):
      rows0[r, pl.ds(d * 16, 16)] = zeros16
    return carry
  lax.fori_loop(0, _C, zbody, 0)

  for r0, rn in _ROW_CHUNKS:
    pltpu.sync_copy(rows0.at[pl.ds(0, rn)], acc.at[pl.ds(t0 + r0, rn)])
  plsc.subcore_barrier()

  rows = (rows0, rows1, rows2, rows3)
  sems = (sem0, sem1, sem2, sem3)
  ssems = (ssem0, ssem1, ssem2, ssem3)

  def scale(rr, k):
    def body(e, cc):
      wv = ew_v[e, pl.ds(k * 16, 16)]
      for d in range(_FH // 16):
        sl = pl.ds(d * 16, 16)
        rr[e, sl] = rr[e, sl] * wv
      return cc
    lax.fori_loop(0, _C, body, 0)

  def sb_body(b, carry):
    sbg = s * _NSB2 + b
    par = lax.rem(b, 2) * _SBCH
    pltpu.sync_copy(src4.at[sbg], src_v)
    pltpu.sync_copy(dst4.at[sbg], dst_v.at[pl.ds(par, _SBCH)])
    pltpu.sync_copy(ewf.at[:, pl.ds(sbg * 64, 64)], ew_v)
    for k in range(_SBCH):
      @pl.when(b > 0)
      def _():
        pltpu.make_async_copy(rows[k], acc.at[dst_v.at[par + k]],
                              ssems[k]).wait()
      pltpu.async_copy(tab_sp.at[src_v.at[k]], rows[k], sems[k])
    for k in range(_SBCH):
      pltpu.make_async_copy(tab_sp.at[src_v.at[k]], rows[k], sems[k]).wait()
      scale(rows[k], k)
      pltpu.async_copy(rows[k], acc.at[dst_v.at[par + k]], ssems[k],
                       add=True)
    return carry
  lax.fori_loop(0, _NSB2, sb_body, 0)
  lastp = ((_NSB2 - 1) % 2) * _SBCH
  for k in range(_SBCH):
    pltpu.make_async_copy(rows[k], acc.at[dst_v.at[lastp + k]],
                          ssems[k]).wait()
  plsc.subcore_barrier()

  for r0, rn in _ROW_CHUNKS:
    rb = t0 + r0
    pltpu.sync_copy(acc.at[pl.ds(rb, rn)], out_s.at[c, pl.ds(rb, rn)])


def _make_sc_feat():
  mesh = plsc.VectorSubcoreMesh(core_axis_name="c", subcore_axis_name="s")
  out_type = jax.ShapeDtypeStruct((_NC, _NA, _FH), jnp.float32)
  scratch = [
      pltpu.VMEM((_SBCH, _C), jnp.int32),
      pltpu.VMEM((2 * _SBCH, _C), jnp.int32),
      pltpu.VMEM((128, 64), jnp.float32),
      pltpu.VMEM((_C, _FH), jnp.float32),
      pltpu.VMEM((_C, _FH), jnp.float32),
      pltpu.VMEM((_C, _FH), jnp.float32),
      pltpu.VMEM((_C, _FH), jnp.float32),
      pltpu.VMEM_SHARED((_N, _FH), jnp.float32),
      pltpu.VMEM_SHARED((_NA, _FH), jnp.float32),
      pltpu.SemaphoreType.DMA,
      pltpu.SemaphoreType.DMA,
      pltpu.SemaphoreType.DMA,
      pltpu.SemaphoreType.DMA,
      pltpu.SemaphoreType.DMA,
      pltpu.SemaphoreType.DMA,
      pltpu.SemaphoreType.DMA,
      pltpu.SemaphoreType.DMA,
  ]
  return pl.kernel(_sc_feat_body, out_type=out_type, mesh=mesh,
                   scratch_types=scratch,
                   compiler_params=pltpu.CompilerParams(
                       use_tc_tiling_on_sc=False))


def _sc_deg_body(dst3, out_d, dst_v, ones_v, z16_v, dacc):
  c = lax.axis_index("c")
  s = lax.axis_index("s")
  wid = s * _NC + c
  t0 = s * _RPT

  zeros16 = jnp.zeros((16,), jnp.float32)
  ones16 = jnp.ones((16,), jnp.float32)

  def zbody(r, carry):
    ones_v[r, :] = ones16
    z16_v[r, :] = zeros16
    return carry
  lax.fori_loop(0, _C, zbody, 0)

  for r0, rn in _ROW_CHUNKS:
    pltpu.sync_copy(z16_v.at[pl.ds(0, rn)], dacc.at[pl.ds(t0 + r0, rn)])
  plsc.subcore_barrier()

  pltpu.sync_copy(dst3.at[wid], dst_v)

  def chunk(j, carry):
    pltpu.sync_copy(ones_v, dacc.at[dst_v.at[j]], add=True)
    return carry
  lax.fori_loop(0, _NCH, chunk, 0)
  plsc.subcore_barrier()

  for r0, rn in _ROW_CHUNKS:
    rb = t0 + r0
    pltpu.sync_copy(dacc.at[pl.ds(rb, rn)], out_d.at[c, pl.ds(rb, rn)])


def _make_sc_deg():
  mesh = plsc.VectorSubcoreMesh(core_axis_name="c", subcore_axis_name="s")
  out_type = jax.ShapeDtypeStruct((_NC, _NA, 16), jnp.float32)
  scratch = [
      pltpu.VMEM((_NCH, _C), jnp.int32),
      pltpu.VMEM((_C, 16), jnp.float32),
      pltpu.VMEM((_C, 16), jnp.float32),
      pltpu.VMEM_SHARED((_NA, 16), jnp.float32),
  ]
  return pl.kernel(_sc_deg_body, out_type=out_type, mesh=mesh,
                   scratch_types=scratch,
                   compiler_params=pltpu.CompilerParams(
                       use_tc_tiling_on_sc=False))


_sc_feat = _make_sc_feat()
_sc_degree = _make_sc_deg()


def _minmax_body(w_ref, mn_ref, mx_ref):
  w = w_ref[...]
  mn_ref[...] = jnp.full((1, 1), jnp.min(w), jnp.float32)
  mx_ref[...] = jnp.full((1, 1), jnp.max(w), jnp.float32)


_minmax = pl.pallas_call(
    _minmax_body,
    out_shape=[jax.ShapeDtypeStruct((1, 1), jnp.float32),
               jax.ShapeDtypeStruct((1, 1), jnp.float32)])

_RB = 2048
_RG = (_EPAD * 16 // 128) // _RB


def _rep_body(mn_ref, mx_ref, w_ref, o_ref):
  i = pl.program_id(0)
  mn = mn_ref[0, 0]
  mx = mx_ref[0, 0]
  eq = mx == mn
  a = jnp.where(eq, 0.0, 1.0 / jnp.where(eq, 1.0, mx - mn))
  b = jnp.where(eq, 1.0, -mn * a)
  w = w_ref[...]
  rr = lax.broadcasted_iota(jnp.int32, (_RB // 16, _RB), 0)
  rc = lax.broadcasted_iota(jnp.int32, (_RB // 16, _RB), 1)
  R = (rc // 16 == rr).astype(jnp.float32)
  rep = lax.dot_general(w, R, (((0,), (0,)), ((), ())),
                        preferred_element_type=jnp.float32)
  r_io = lax.broadcasted_iota(jnp.int32, (128, _RB), 0)
  c_io = lax.broadcasted_iota(jnp.int32, (128, _RB), 1)
  e = (_RB * 8) * i + 128 * (c_io // 16) + r_io
  o_ref[...] = jnp.where(e < _E, rep * a + b, 0.0)


_rep = pl.pallas_call(
    _rep_body,
    grid=(_RG,),
    in_specs=[
        pl.BlockSpec((1, 1), lambda i: (0, 0)),
        pl.BlockSpec((1, 1), lambda i: (0, 0)),
        pl.BlockSpec((_RB // 16, 128), lambda i: (i, 0)),
    ],
    out_specs=pl.BlockSpec((128, _RB), lambda i: (0, i)),
    out_shape=jax.ShapeDtypeStruct((128, _EPAD * 16 // 128), jnp.float32))


_BLK = 400
_NBLK = _N // _BLK


def _dense0_body(sp, dp, x, wl0, bl0, wr0, gamma, beta, rm, rv, wl1,
                 h_out, g_out):
  ssum = jnp.concatenate([sp[0], sp[1]], axis=1)
  dsum = dp[0] + dp[1]
  deg = jnp.clip(dsum[:, 0:1], 1.0, None)
  aggr = ssum / deg
  dn = (((1,), (1,)), ((), ()))
  pre = (lax.dot_general(aggr, wl0[...], dn, preferred_element_type=jnp.float32)
         + bl0[...]
         + lax.dot_general(x[...], wr0[...], dn, preferred_element_type=jnp.float32))
  inv = lax.rsqrt(rv[...] + 1e-5)
  hh = jnp.maximum((pre - rm[...]) * inv * gamma[...] + beta[...], 0.0)
  h_out[...] = hh
  g_out[...] = lax.dot_general(hh, wl1[...], dn, preferred_element_type=jnp.float32)


_dense0 = pl.pallas_call(
    _dense0_body,
    grid=(_NBLK,),
    in_specs=[
        pl.BlockSpec((_NC, _BLK, _FH), lambda i: (0, i, 0)),
        pl.BlockSpec((_NC, _BLK, 16), lambda i: (0, i, 0)),
        pl.BlockSpec((_BLK, _IN), lambda i: (i, 0)),
        pl.BlockSpec((_H, _IN), lambda i: (0, 0)),
        pl.BlockSpec((1, _H), lambda i: (0, 0)),
        pl.BlockSpec((_H, _IN), lambda i: (0, 0)),
        pl.BlockSpec((1, _H), lambda i: (0, 0)),
        pl.BlockSpec((1, _H), lambda i: (0, 0)),
        pl.BlockSpec((1, _H), lambda i: (0, 0)),
        pl.BlockSpec((1, _H), lambda i: (0, 0)),
        pl.BlockSpec((_OUT, _H), lambda i: (0, 0)),
    ],
    out_specs=[
        pl.BlockSpec((_BLK, _H), lambda i: (i, 0)),
        pl.BlockSpec((_BLK, _OUT), lambda i: (i, 0)),
    ],
    out_shape=[
        jax.ShapeDtypeStruct((_N, _H), jnp.float32),
        jax.ShapeDtypeStruct((_N, _OUT), jnp.float32),
    ])


def _dense1_body(sp, dp, h, wr1, bl1, lsm_out, var_out, acc_s):
  i = pl.program_id(0)
  ssum = jnp.concatenate([sp[0], sp[1]], axis=1)
  dsum = dp[0] + dp[1]
  deg = jnp.clip(dsum[:, 0:1], 1.0, None)
  dn = (((1,), (1,)), ((), ()))
  o = (ssum / deg + bl1[...]
       + lax.dot_general(h[...], wr1[...], dn, preferred_element_type=jnp.float32))
  m = jnp.max(o, axis=1, keepdims=True)
  lse = jnp.log(jnp.sum(jnp.exp(o - m), axis=1, keepdims=True)) + m
  lsm_out[...] = o - lse
  bs = jnp.sum(o)
  bss = jnp.sum(o * o)

  @pl.when(i == 0)
  def _():
    acc_s[0] = bs
    acc_s[1] = bss

  @pl.when(i > 0)
  def _():
    acc_s[0] = acc_s[0] + bs
    acc_s[1] = acc_s[1] + bss

  tot = float(_N * _OUT)
  var_out[...] = jnp.full((1, 1), (acc_s[1] - acc_s[0] * acc_s[0] / tot)
                          / (tot - 1.0), jnp.float32)


_dense1 = pl.pallas_call(
    _dense1_body,
    grid=(_NBLK,),
    in_specs=[
        pl.BlockSpec((_NC, _BLK, _FH), lambda i: (0, i, 0)),
        pl.BlockSpec((_NC, _BLK, 16), lambda i: (0, i, 0)),
        pl.BlockSpec((_BLK, _H), lambda i: (i, 0)),
        pl.BlockSpec((_OUT, _H), lambda i: (0, 0)),
        pl.BlockSpec((1, _OUT), lambda i: (0, 0)),
    ],
    out_specs=[
        pl.BlockSpec((_BLK, _OUT), lambda i: (i, 0)),
        pl.BlockSpec((1, 1), lambda i: (0, 0)),
    ],
    out_shape=[
        jax.ShapeDtypeStruct((_N, _OUT), jnp.float32),
        jax.ShapeDtypeStruct((1, 1), jnp.float32),
    ],
    scratch_shapes=[pltpu.SMEM((2,), jnp.float32)])


def kernel(x, edge_index, edge_weight, Wl0, bl0, Wr0, gamma, beta,
           running_mean, running_var, Wl1, bl1, Wr1):
  pad = _EPAD - _E
  mn, mx = _minmax(edge_weight.reshape(_E // 128, 128))
  wraw = jnp.concatenate([edge_weight, jnp.zeros((pad,), jnp.float32)]
                         ).reshape(_EPAD // 128, 128)
  ewf = _rep(mn, mx, wraw)
  src4 = jnp.concatenate([edge_index[0], jnp.zeros((pad,), jnp.int32)]
                         ).reshape(_TSB, _SBCH, _C)
  junk = _N + jnp.arange(pad, dtype=jnp.int32) % (_NA - _N)
  dst_p = jnp.concatenate([edge_index[1], junk])
  dst3 = dst_p.reshape(_NW, _NCH, _C)
  dst4 = dst_p.reshape(_TSB, _SBCH, _C)

  dp0 = _sc_degree(dst3)
  xh = x.reshape(_N, 2, _FH).transpose(1, 0, 2)
  sp0 = _sc_feat(xh, src4, dst4, ewf)
  h, g = _dense0(sp0, dp0, x, Wl0, bl0.reshape(1, -1), Wr0,
                 gamma.reshape(1, -1), beta.reshape(1, -1),
                 running_mean.reshape(1, -1), running_var.reshape(1, -1), Wl1)
  gh = g.reshape(_N, 2, _FH).transpose(1, 0, 2)
  sp1 = _sc_feat(gh, src4, dst4, ewf)
  lsm, var = _dense1(sp1, dp0, h, Wr1, bl1.reshape(1, -1))
  return lsm, var.reshape(())
